# baseline (speedup 1.0000x reference)
import functools

import jax
import jax.numpy as jnp
from jax import lax
from jax.experimental import pallas as pl
from jax.experimental.pallas import tpu as pltpu
from jax.experimental.pallas import tpu_sc as plsc

N = 10000
E = 320000
NODE = 128
EDGE = 16
TIME = 32
MEM = 128
HEADS = 4
HPD = 32
MSG = NODE + EDGE + TIME
L0_IN = NODE + MEM + TIME

BLK = 1000
NW = 32
EPW = E // NW
KW = 80
NWIN = EPW // KW
NPAD = 10240
NPS = NPAD // 16
WX = 144
SS0 = NODE
DN0 = NODE + HEADS


def _pack_ext(hh, s):
    b = hh.shape[0]
    return jnp.concatenate(
        [hh, s[:, :HEADS], jnp.ones((b, HEADS), jnp.float32),
         jnp.zeros((b, WX - DN0 - HEADS), jnp.float32)], axis=1)


def _pack_sd(s):
    b = s.shape[0]
    return jnp.concatenate(
        [s[:, HEADS:], jnp.zeros((b, 12), jnp.float32)], axis=1)


def _dense1_body(nf_ref, nfd2_ref, ea_ref, tt_ref, has_ref, tw_ref, tb_ref,
                 w1a_ref, w1c_ref, w1d_ref, w1e_ref, b1_ref, w2_ref, b2_ref,
                 wiht_ref, bih_ref, bhh_ref, w0_ref, a0_ref,
                 hhx_ref, sd_ref):
    nf = nf_ref[...]
    t = tt_ref[...] * has_ref[...]
    te = jnp.sin(tw_ref[...] * t + tb_ref[...])
    u = (nf @ w1a_ref[...] + nfd2_ref[...] @ w1c_ref[...]
         + ea_ref[...] @ w1d_ref[...] + te @ w1e_ref[...] + b1_ref[...])
    u = jnp.maximum(u, 0.0)
    msgs = u @ w2_ref[...] + b2_ref[...]
    gi = msgs @ wiht_ref[...] + bih_ref[...]
    gh = bhh_ref[...]
    ir, iz, inn = gi[:, :MEM], gi[:, MEM:2 * MEM], gi[:, 2 * MEM:]
    hr, hz, hn = gh[:, :MEM], gh[:, MEM:2 * MEM], gh[:, 2 * MEM:]
    r = jax.nn.sigmoid(ir + hr)
    z = jax.nn.sigmoid(iz + hz)
    nn_ = jnp.tanh(inn + r * hn)
    newh = (1.0 - z) * nn_
    memv = newh * has_ref[...]
    x = jnp.concatenate([nf, memv, te], axis=1)
    hh0 = x @ w0_ref[...]
    s0 = hh0 @ a0_ref[...]
    hhx_ref[...] = _pack_ext(hh0, s0)
    sd_ref[...] = _pack_sd(s0)


def _unpack_finish(a0_ref, a1_ref, b_ref):
    acc = a0_ref[...] + a1_ref[...]
    parts = []
    for h in range(HEADS):
        parts.append(acc[:, h * HPD:(h + 1) * HPD]
                     / (acc[:, DN0 + h:DN0 + h + 1] + 1e-16))
    return jnp.concatenate(parts, axis=1) + b_ref[...]


def _finish_proj_body(a0_ref, a1_ref, b_ref, w_ref, am_ref, hhx_ref, sd_ref):
    x = _unpack_finish(a0_ref, a1_ref, b_ref)
    hh = x @ w_ref[...]
    s = hh @ am_ref[...]
    hhx_ref[...] = _pack_ext(hh, s)
    sd_ref[...] = _pack_sd(s)


def _finish_cls_body(a0_ref, a1_ref, b_ref, w1_ref, b1_ref, w2_ref, b2_ref,
                     out_ref):
    x = _unpack_finish(a0_ref, a1_ref, b_ref)
    u = jnp.maximum(x @ w1_ref[...] + b1_ref[...], 0.0)
    out_ref[...] = u @ w2_ref[...] + b2_ref[...]


def _full(shape):
    return pl.BlockSpec(shape, lambda i: (0,) * len(shape))


def _rows(width):
    return pl.BlockSpec((BLK, width), lambda i: (i, 0))


def _bcast_lane(vec, j):
    return lax.gather(
        vec, jnp.full((16, 1), j, jnp.int32),
        lax.GatherDimensionNumbers(offset_dims=(), collapsed_slice_dims=(0,),
                                   start_index_map=(0,)),
        (1,), mode=lax.GatherScatterMode.PROMISE_IN_BOUNDS)


def _shift4(vec):
    idx = jnp.maximum(lax.iota(jnp.int32, 16) - 4, 0)
    return lax.gather(
        vec, idx[:, None],
        lax.GatherDimensionNumbers(offset_dims=(), collapsed_slice_dims=(0,),
                                   start_index_map=(0,)),
        (1,), mode=lax.GatherScatterMode.PROMISE_IN_BOUNDS)


def _gat_sc_body(src_ref, dst_ref, hhx_ref, sd_ref, z_ref,
                 acc_ref,
                 idx_s, idx_d, rows, sdb, out_sp, sem):
    cid = lax.axis_index("c")
    sid = lax.axis_index("s")
    wid = sid * 2 + cid

    pltpu.sync_copy(z_ref.at[pl.ds(sid * NPS, NPS)],
                    out_sp.at[pl.ds(sid * NPS, NPS)])
    plsc.subcore_barrier()

    base_edge = wid * EPW

    def window(w, carry):
        wb = base_edge + w * KW
        pltpu.sync_copy(src_ref.at[pl.ds(wb, KW)], idx_s)
        pltpu.sync_copy(dst_ref.at[pl.ds(wb, KW)], idx_d)
        cp1 = pltpu.async_copy(hhx_ref.at[idx_s], rows, sem)
        cp2 = pltpu.async_copy(sd_ref.at[idx_d], sdb, sem)
        cp1.wait()
        cp2.wait()
        for j in range(KW):
            sseg = rows[j, pl.ds(SS0, 16)]
            v = sseg + sdb[j, :]
            v = jnp.where(v >= 0.0, v, 0.2 * v)
            w16 = jnp.exp(v)
            rows[j, pl.ds(SS0, 16)] = sseg * _shift4(w16)
            for h in range(HEADS):
                wh = _bcast_lane(w16, h)
                c0 = h * HPD
                rows[j, pl.ds(c0, 16)] = rows[j, pl.ds(c0, 16)] * wh
                rows[j, pl.ds(c0 + 16, 16)] = rows[j, pl.ds(c0 + 16, 16)] * wh
        pltpu.sync_copy(rows, out_sp.at[idx_d], add=True)
        return carry

    lax.fori_loop(0, NWIN, window, 0)
    plsc.subcore_barrier()
    pltpu.sync_copy(out_sp.at[pl.ds(sid * NPS, NPS)],
                    acc_ref.at[cid, pl.ds(sid * NPS, NPS)])


def _gat_aggregate(src, dst, hhx, sd, z):
    mesh = plsc.VectorSubcoreMesh(core_axis_name="c", subcore_axis_name="s")
    f = pl.kernel(
        _gat_sc_body,
        mesh=mesh,
        compiler_params=pltpu.CompilerParams(use_tc_tiling_on_sc=False,
                                             needs_layout_passes=False),
        out_type=jax.ShapeDtypeStruct((2, NPAD, WX), jnp.float32),
        scratch_types=[
            pltpu.VMEM((KW,), jnp.int32),
            pltpu.VMEM((KW,), jnp.int32),
            pltpu.VMEM((KW, WX), jnp.float32),
            pltpu.VMEM((KW, 16), jnp.float32),
            pltpu.VMEM_SHARED((NPAD, WX), jnp.float32),
            pltpu.SemaphoreType.DMA,
        ],
    )
    return f(src, dst, hhx, sd, z)


def _shift_up1(vec):
    idx = jnp.minimum(lax.iota(jnp.int32, 16) + 1, 15)
    return lax.gather(
        vec, idx[:, None],
        lax.GatherDimensionNumbers(offset_dims=(), collapsed_slice_dims=(0,),
                                   start_index_map=(0,)),
        (1,), mode=lax.GatherScatterMode.PROMISE_IN_BOUNDS)


EPT = E // 16
NSL = NPAD // 16
NCH = NSL // 128


def _last_sc_body(src_ref, dst_ref, et_ref, ea_ref, nf_ref,
                  le_ref, tt_ref, eag_ref, nfd2_ref,
                  lastl, sbuf, mrg, leraw, lec, d2b, ttb, eab, nfb,
                  shared, sem):
    cid = lax.axis_index("c")
    sid = lax.axis_index("s")
    iota = lax.iota(jnp.int32, 16)

    @pl.when(cid == 0)
    def _():
        def initb(i, c):
            lastl[pl.ds(i * 16, 16)] = jnp.full((16,), -1, jnp.int32)
            return c
        lax.fori_loop(0, NPAD // 16, initb, 0)

        base = sid * EPT

        def scan(i, c):
            eb = base + i * 16
            pltpu.sync_copy(src_ref.at[pl.ds(eb, 16)], sbuf)
            sv = sbuf[...]
            key = sv * 16 + iota
            eidv = jnp.full((16,), eb, jnp.int32) + iota
            ks, vs = plsc.sort_key_val(key, eidv)
            srcs = lax.shift_right_logical(ks, 4)
            nxt = _shift_up1(srcs)
            msk = jnp.logical_or(srcs != nxt, iota == 15)
            plsc.store_scatter(lastl, [srcs], vs, mask=msk)
            return c
        lax.fori_loop(0, EPT // 16, scan, 0)

        pltpu.sync_copy(lastl, shared.at[sid])
        plsc.subcore_barrier()
        pltpu.sync_copy(shared.at[:, pl.ds(sid * NSL, NSL)], mrg)

        def red(q, c):
            acc = mrg[0, pl.ds(q * 16, 16)]
            for r in range(1, 16):
                acc = jnp.maximum(acc, mrg[r, pl.ds(q * 16, 16)])
            leraw[pl.ds(q * 16, 16)] = acc
            lec[(q * 16) // 128, pl.ds((q * 16) % 128, 16)] = (
                jnp.maximum(acc, 0))
            return c
        lax.fori_loop(0, NSL // 16, red, 0)

        pltpu.sync_copy(leraw, le_ref.at[pl.ds(sid * NSL, NSL)])
        for k in range(NCH):
            pltpu.async_copy(dst_ref.at[lec.at[k]], d2b.at[k], sem).wait()
            pltpu.async_copy(et_ref.at[lec.at[k]], ttb.at[k], sem).wait()
            cpa = pltpu.async_copy(
                ea_ref.at[lec.at[k]], eab.at[pl.ds(k * 128, 128)], sem)
            cpb = pltpu.async_copy(
                nf_ref.at[d2b.at[k]], nfb.at[pl.ds(k * 128, 128)], sem)
            cpa.wait()
            cpb.wait()
            pltpu.sync_copy(ttb.at[k],
                            tt_ref.at[pl.ds(sid * NSL + k * 128, 128)])
        pltpu.sync_copy(eab, eag_ref.at[pl.ds(sid * NSL, NSL)])
        pltpu.sync_copy(nfb, nfd2_ref.at[pl.ds(sid * NSL, NSL)])


def _last_edge(src, dst, et, ea, nf):
    mesh = plsc.VectorSubcoreMesh(core_axis_name="c", subcore_axis_name="s")
    f = pl.kernel(
        _last_sc_body,
        mesh=mesh,
        compiler_params=pltpu.CompilerParams(use_tc_tiling_on_sc=False,
                                             needs_layout_passes=False),
        out_type=[
            jax.ShapeDtypeStruct((NPAD,), jnp.int32),
            jax.ShapeDtypeStruct((NPAD,), jnp.float32),
            jax.ShapeDtypeStruct((NPAD, EDGE), jnp.float32),
            jax.ShapeDtypeStruct((NPAD, NODE), jnp.float32),
        ],
        scratch_types=[
            pltpu.VMEM((NPAD,), jnp.int32),
            pltpu.VMEM((16,), jnp.int32),
            pltpu.VMEM((16, NSL), jnp.int32),
            pltpu.VMEM((NSL,), jnp.int32),
            pltpu.VMEM((NCH, 128), jnp.int32),
            pltpu.VMEM((NCH, 128), jnp.int32),
            pltpu.VMEM((NCH, 128), jnp.float32),
            pltpu.VMEM((NSL, EDGE), jnp.float32),
            pltpu.VMEM((NSL, NODE), jnp.float32),
            pltpu.VMEM_SHARED((16, NPAD), jnp.int32),
            pltpu.SemaphoreType.DMA,
        ],
    )
    return f(src, dst, et, ea, nf)


def kernel(node_features, edge_index, edge_attr, edge_times, time_w, time_b,
           msg_W1, msg_b1, msg_W2, msg_b2, gru_Wih, gru_Whh, gru_bih, gru_bhh,
           gat0_W, gat0_asrc, gat0_adst, gat0_b,
           gat1_W, gat1_asrc, gat1_adst, gat1_b,
           cls_W1, cls_b1, cls_W2, cls_b2):
    src = edge_index[0]
    dst = edge_index[1]
    lep, ttp, eap, nfd2p = _last_edge(src, dst, edge_times, edge_attr,
                                      node_features)
    has_f = (lep[:N] >= 0).astype(jnp.float32)[:, None]

    w1a = msg_W1[:NODE]
    w1c = msg_W1[NODE + MEM:NODE + MEM + NODE]
    w1d = msg_W1[NODE + MEM + NODE:NODE + MEM + NODE + EDGE]
    w1e = msg_W1[NODE + MEM + NODE + EDGE:]
    wiht = gru_Wih.T
    w0cat = jnp.transpose(gat0_W, (1, 0, 2)).reshape(L0_IN, HEADS * HPD)
    w1cat = jnp.transpose(gat1_W, (1, 0, 2)).reshape(NODE, HEADS * HPD)

    def amat(asrc, adst):
        a = jnp.zeros((HEADS * HPD, 2 * HEADS), jnp.float32)
        for h in range(HEADS):
            a = a.at[h * HPD:(h + 1) * HPD, h].set(asrc[h])
            a = a.at[h * HPD:(h + 1) * HPD, HEADS + h].set(adst[h])
        return a

    a0 = amat(gat0_asrc, gat0_adst)
    a1 = amat(gat1_asrc, gat1_adst)
    b0flat = gat0_b.reshape(1, -1)
    b1flat = gat1_b.reshape(1, -1)
    z = jnp.zeros((NPAD, WX), jnp.float32)

    grid = (N // BLK,)
    hhx0, sd0 = pl.pallas_call(
        _dense1_body,
        grid=grid,
        in_specs=[
            _rows(NODE), _rows(NODE), _rows(EDGE), _rows(1), _rows(1),
            _full((1, TIME)), _full((1, TIME)),
            _full((NODE, MSG)), _full((NODE, MSG)), _full((EDGE, MSG)),
            _full((TIME, MSG)), _full((1, MSG)), _full((MSG, MSG)),
            _full((1, MSG)), _full((MSG, 3 * MEM)), _full((1, 3 * MEM)),
            _full((1, 3 * MEM)), _full((L0_IN, HEADS * HPD)),
            _full((HEADS * HPD, 2 * HEADS)),
        ],
        out_specs=[_rows(WX), _rows(16)],
        out_shape=[
            jax.ShapeDtypeStruct((N, WX), jnp.float32),
            jax.ShapeDtypeStruct((N, 16), jnp.float32),
        ],
    )(node_features, nfd2p, eap, ttp[:, None], has_f,
      time_w[None, :], time_b[None, :], w1a, w1c, w1d, w1e,
      msg_b1[None, :], msg_W2, msg_b2[None, :],
      wiht, gru_bih[None, :], gru_bhh[None, :], w0cat, a0)

    acc0 = _gat_aggregate(src, dst, hhx0, sd0, z)

    hhx1, sd1 = pl.pallas_call(
        _finish_proj_body,
        grid=grid,
        in_specs=[_rows(WX), _rows(WX), _full((1, HEADS * HPD)),
                  _full((NODE, HEADS * HPD)), _full((HEADS * HPD, 2 * HEADS))],
        out_specs=[_rows(WX), _rows(16)],
        out_shape=[
            jax.ShapeDtypeStruct((N, WX), jnp.float32),
            jax.ShapeDtypeStruct((N, 16), jnp.float32),
        ],
    )(acc0[0], acc0[1], b0flat, w1cat, a1)

    acc1 = _gat_aggregate(src, dst, hhx1, sd1, z)

    logits = pl.pallas_call(
        _finish_cls_body,
        grid=grid,
        in_specs=[_rows(WX), _rows(WX), _full((1, HEADS * HPD)),
                  _full((NODE, NODE // 2)), _full((1, NODE // 2)),
                  _full((NODE // 2, 1)), _full((1, 1))],
        out_specs=_rows(1),
        out_shape=jax.ShapeDtypeStruct((N, 1), jnp.float32),
    )(acc1[0], acc1[1], b1flat,
      cls_W1, cls_b1[None, :], cls_W2, cls_b2[None, :])

    return logits


# speedup vs baseline: 1.4227x; 1.4227x over previous
import functools

import jax
import jax.numpy as jnp
from jax import lax
from jax.experimental import pallas as pl
from jax.experimental.pallas import tpu as pltpu
from jax.experimental.pallas import tpu_sc as plsc

N = 10000
E = 320000
NODE = 128
EDGE = 16
TIME = 32
MEM = 128
HEADS = 4
HPD = 32
MSG = NODE + EDGE + TIME
L0_IN = NODE + MEM + TIME

BLK = 1000
NW = 32
EPW = E // NW
KW = 80
NWIN = EPW // KW
NPAD = 10240
NPS = NPAD // 16
WX = 144
SS0 = NODE
DN0 = NODE + HEADS


def _pack_ext(hh, s):
    b = hh.shape[0]
    return jnp.concatenate(
        [hh, s[:, :HEADS], jnp.ones((b, HEADS), jnp.float32),
         jnp.zeros((b, WX - DN0 - HEADS), jnp.float32)], axis=1)


def _pack_sd(s):
    b = s.shape[0]
    return jnp.concatenate(
        [s[:, HEADS:], jnp.zeros((b, 12), jnp.float32)], axis=1)


def _dense1_body(nf_ref, nfd2_ref, ea_ref, tt_ref, has_ref, tw_ref, tb_ref,
                 w1a_ref, w1c_ref, w1d_ref, w1e_ref, b1_ref, w2_ref, b2_ref,
                 wiht_ref, bih_ref, bhh_ref, w0_ref, a0_ref,
                 hhx_ref, sd_ref):
    nf = nf_ref[...]
    t = tt_ref[...] * has_ref[...]
    te = jnp.sin(tw_ref[...] * t + tb_ref[...])
    u = (nf @ w1a_ref[...] + nfd2_ref[...] @ w1c_ref[...]
         + ea_ref[...] @ w1d_ref[...] + te @ w1e_ref[...] + b1_ref[...])
    u = jnp.maximum(u, 0.0)
    msgs = u @ w2_ref[...] + b2_ref[...]
    gi = msgs @ wiht_ref[...] + bih_ref[...]
    gh = bhh_ref[...]
    ir, iz, inn = gi[:, :MEM], gi[:, MEM:2 * MEM], gi[:, 2 * MEM:]
    hr, hz, hn = gh[:, :MEM], gh[:, MEM:2 * MEM], gh[:, 2 * MEM:]
    r = jax.nn.sigmoid(ir + hr)
    z = jax.nn.sigmoid(iz + hz)
    nn_ = jnp.tanh(inn + r * hn)
    newh = (1.0 - z) * nn_
    memv = newh * has_ref[...]
    x = jnp.concatenate([nf, memv, te], axis=1)
    hh0 = x @ w0_ref[...]
    s0 = hh0 @ a0_ref[...]
    hhx_ref[...] = _pack_ext(hh0, s0)
    sd_ref[...] = _pack_sd(s0)


def _unpack_finish(a0_ref, a1_ref, b_ref):
    acc = a0_ref[...] + a1_ref[...]
    parts = []
    for h in range(HEADS):
        parts.append(acc[:, h * HPD:(h + 1) * HPD]
                     / (acc[:, DN0 + h:DN0 + h + 1] + 1e-16))
    return jnp.concatenate(parts, axis=1) + b_ref[...]


def _finish_proj_body(a0_ref, a1_ref, b_ref, w_ref, am_ref, hhx_ref, sd_ref):
    x = _unpack_finish(a0_ref, a1_ref, b_ref)
    hh = x @ w_ref[...]
    s = hh @ am_ref[...]
    hhx_ref[...] = _pack_ext(hh, s)
    sd_ref[...] = _pack_sd(s)


def _finish_cls_body(a0_ref, a1_ref, b_ref, w1_ref, b1_ref, w2_ref, b2_ref,
                     out_ref):
    x = _unpack_finish(a0_ref, a1_ref, b_ref)
    u = jnp.maximum(x @ w1_ref[...] + b1_ref[...], 0.0)
    out_ref[...] = u @ w2_ref[...] + b2_ref[...]


def _full(shape):
    return pl.BlockSpec(shape, lambda i: (0,) * len(shape))


def _rows(width):
    return pl.BlockSpec((BLK, width), lambda i: (i, 0))


def _bcast_lane(vec, j):
    return lax.gather(
        vec, jnp.full((16, 1), j, jnp.int32),
        lax.GatherDimensionNumbers(offset_dims=(), collapsed_slice_dims=(0,),
                                   start_index_map=(0,)),
        (1,), mode=lax.GatherScatterMode.PROMISE_IN_BOUNDS)


def _shift4(vec):
    idx = jnp.maximum(lax.iota(jnp.int32, 16) - 4, 0)
    return lax.gather(
        vec, idx[:, None],
        lax.GatherDimensionNumbers(offset_dims=(), collapsed_slice_dims=(0,),
                                   start_index_map=(0,)),
        (1,), mode=lax.GatherScatterMode.PROMISE_IN_BOUNDS)


def _gat_sc_body(src_ref, dst_ref, hhx_ref, sd_ref, z_ref,
                 acc_ref,
                 idx_s, idx_d, rows, sdb, out_sp, sem):
    cid = lax.axis_index("c")
    sid = lax.axis_index("s")
    wid = sid * 2 + cid

    pltpu.sync_copy(z_ref.at[pl.ds(sid * NPS, NPS)],
                    out_sp.at[pl.ds(sid * NPS, NPS)])
    plsc.subcore_barrier()

    base_edge = wid * EPW

    def window(w, carry):
        wb = base_edge + w * KW
        pltpu.sync_copy(src_ref.at[pl.ds(wb, KW)], idx_s)
        pltpu.sync_copy(dst_ref.at[pl.ds(wb, KW)], idx_d)
        cp1 = pltpu.async_copy(hhx_ref.at[idx_s], rows, sem)
        cp2 = pltpu.async_copy(sd_ref.at[idx_d], sdb, sem)
        cp1.wait()
        cp2.wait()
        for j in range(KW):
            sseg = rows[j, pl.ds(SS0, 16)]
            v = sseg + sdb[j, :]
            v = jnp.where(v >= 0.0, v, 0.2 * v)
            w16 = jnp.exp(v)
            rows[j, pl.ds(SS0, 16)] = sseg * _shift4(w16)
            for h in range(HEADS):
                wh = _bcast_lane(w16, h)
                c0 = h * HPD
                rows[j, pl.ds(c0, 16)] = rows[j, pl.ds(c0, 16)] * wh
                rows[j, pl.ds(c0 + 16, 16)] = rows[j, pl.ds(c0 + 16, 16)] * wh
        pltpu.sync_copy(rows, out_sp.at[idx_d], add=True)
        return carry

    lax.fori_loop(0, NWIN, window, 0)
    plsc.subcore_barrier()
    pltpu.sync_copy(out_sp.at[pl.ds(sid * NPS, NPS)],
                    acc_ref.at[cid, pl.ds(sid * NPS, NPS)])


def _gat_aggregate(src, dst, hhx, sd, z):
    mesh = plsc.VectorSubcoreMesh(core_axis_name="c", subcore_axis_name="s")
    f = pl.kernel(
        _gat_sc_body,
        mesh=mesh,
        compiler_params=pltpu.CompilerParams(use_tc_tiling_on_sc=False,
                                             needs_layout_passes=False),
        out_type=jax.ShapeDtypeStruct((2, NPAD, WX), jnp.float32),
        scratch_types=[
            pltpu.VMEM((KW,), jnp.int32),
            pltpu.VMEM((KW,), jnp.int32),
            pltpu.VMEM((KW, WX), jnp.float32),
            pltpu.VMEM((KW, 16), jnp.float32),
            pltpu.VMEM_SHARED((NPAD, WX), jnp.float32),
            pltpu.SemaphoreType.DMA,
        ],
    )
    return f(src, dst, hhx, sd, z)


def _shift_up1(vec):
    idx = jnp.minimum(lax.iota(jnp.int32, 16) + 1, 15)
    return lax.gather(
        vec, idx[:, None],
        lax.GatherDimensionNumbers(offset_dims=(), collapsed_slice_dims=(0,),
                                   start_index_map=(0,)),
        (1,), mode=lax.GatherScatterMode.PROMISE_IN_BOUNDS)


EPT = E // 16
SCH = 4000
NSL = NPAD // 16
NCH = NSL // 128


def _last_sc_body(src_ref, dst_ref, et_ref, ea_ref, nf_ref,
                  le_ref, tt_ref, eag_ref, nfd2_ref,
                  lastl, sbuf, mrg, leraw, lec, d2b, ttb, eab, nfb,
                  shared, sem):
    cid = lax.axis_index("c")
    sid = lax.axis_index("s")
    iota = lax.iota(jnp.int32, 16)

    @pl.when(cid == 0)
    def _():
        def initb(i, c):
            lastl[pl.ds(i * 16, 16)] = jnp.full((16,), -1, jnp.int32)
            return c
        lax.fori_loop(0, NPAD // 16, initb, 0)

        base = sid * EPT

        def chunk(ci, c0):
            pltpu.sync_copy(src_ref.at[pl.ds(base + ci * SCH, SCH)], sbuf)

            def scan(i, c):
                sv = sbuf[pl.ds(i * 16, 16)]
                key = sv * 16 + iota
                eidv = jnp.full((16,), base + ci * SCH, jnp.int32) + i * 16 + iota
                ks, vs = plsc.sort_key_val(key, eidv)
                srcs = lax.shift_right_logical(ks, 4)
                nxt = _shift_up1(srcs)
                msk = jnp.logical_or(srcs != nxt, iota == 15)
                plsc.store_scatter(lastl, [srcs], vs, mask=msk)
                return c
            lax.fori_loop(0, SCH // 16, scan, 0)
            return c0
        lax.fori_loop(0, EPT // SCH, chunk, 0)

        pltpu.sync_copy(lastl, shared.at[sid])
        plsc.subcore_barrier()
        for k in range(NCH):
            pltpu.sync_copy(shared.at[:, pl.ds(sid * NSL + k * 128, 128)], mrg)

            def red(q, c):
                acc = mrg[0, pl.ds(q * 16, 16)]
                for r in range(1, 16):
                    acc = jnp.maximum(acc, mrg[r, pl.ds(q * 16, 16)])
                leraw[pl.ds(k * 128 + q * 16, 16)] = acc
                lec[k, pl.ds(q * 16, 16)] = jnp.maximum(acc, 0)
                return c
            lax.fori_loop(0, 8, red, 0)

        pltpu.sync_copy(leraw, le_ref.at[pl.ds(sid * NSL, NSL)])
        cps = [pltpu.async_copy(dst_ref.at[lec.at[k]], d2b.at[k], sem)
               for k in range(NCH)]
        for cp in cps:
            cp.wait()
        cps = []
        for k in range(NCH):
            cps.append(pltpu.async_copy(et_ref.at[lec.at[k]], ttb.at[k], sem))
            cps.append(pltpu.async_copy(
                ea_ref.at[lec.at[k]], eab.at[pl.ds(k * 128, 128)], sem))
            cps.append(pltpu.async_copy(
                nf_ref.at[d2b.at[k]], nfb.at[pl.ds(k * 128, 128)], sem))
        for cp in cps:
            cp.wait()
        for k in range(NCH):
            pltpu.sync_copy(ttb.at[k],
                            tt_ref.at[pl.ds(sid * NSL + k * 128, 128)])
        pltpu.sync_copy(eab, eag_ref.at[pl.ds(sid * NSL, NSL)])
        pltpu.sync_copy(nfb, nfd2_ref.at[pl.ds(sid * NSL, NSL)])


def _last_edge(src, dst, et, ea, nf):
    mesh = plsc.VectorSubcoreMesh(core_axis_name="c", subcore_axis_name="s")
    f = pl.kernel(
        _last_sc_body,
        mesh=mesh,
        compiler_params=pltpu.CompilerParams(use_tc_tiling_on_sc=False,
                                             needs_layout_passes=False),
        out_type=[
            jax.ShapeDtypeStruct((NPAD,), jnp.int32),
            jax.ShapeDtypeStruct((NPAD,), jnp.float32),
            jax.ShapeDtypeStruct((NPAD, EDGE), jnp.float32),
            jax.ShapeDtypeStruct((NPAD, NODE), jnp.float32),
        ],
        scratch_types=[
            pltpu.VMEM((NPAD,), jnp.int32),
            pltpu.VMEM((SCH,), jnp.int32),
            pltpu.VMEM((16, 128), jnp.int32),
            pltpu.VMEM((NSL,), jnp.int32),
            pltpu.VMEM((NCH, 128), jnp.int32),
            pltpu.VMEM((NCH, 128), jnp.int32),
            pltpu.VMEM((NCH, 128), jnp.float32),
            pltpu.VMEM((NSL, EDGE), jnp.float32),
            pltpu.VMEM((NSL, NODE), jnp.float32),
            pltpu.VMEM_SHARED((16, NPAD), jnp.int32),
            pltpu.SemaphoreType.DMA,
        ],
    )
    return f(src, dst, et, ea, nf)


def kernel(node_features, edge_index, edge_attr, edge_times, time_w, time_b,
           msg_W1, msg_b1, msg_W2, msg_b2, gru_Wih, gru_Whh, gru_bih, gru_bhh,
           gat0_W, gat0_asrc, gat0_adst, gat0_b,
           gat1_W, gat1_asrc, gat1_adst, gat1_b,
           cls_W1, cls_b1, cls_W2, cls_b2):
    src = edge_index[0]
    dst = edge_index[1]
    lep, ttp, eap, nfd2p = _last_edge(src, dst, edge_times, edge_attr,
                                      node_features)
    has_f = (lep[:N] >= 0).astype(jnp.float32)[:, None]

    w1a = msg_W1[:NODE]
    w1c = msg_W1[NODE + MEM:NODE + MEM + NODE]
    w1d = msg_W1[NODE + MEM + NODE:NODE + MEM + NODE + EDGE]
    w1e = msg_W1[NODE + MEM + NODE + EDGE:]
    wiht = gru_Wih.T
    w0cat = jnp.transpose(gat0_W, (1, 0, 2)).reshape(L0_IN, HEADS * HPD)
    w1cat = jnp.transpose(gat1_W, (1, 0, 2)).reshape(NODE, HEADS * HPD)

    def amat(asrc, adst):
        a = jnp.zeros((HEADS * HPD, 2 * HEADS), jnp.float32)
        for h in range(HEADS):
            a = a.at[h * HPD:(h + 1) * HPD, h].set(asrc[h])
            a = a.at[h * HPD:(h + 1) * HPD, HEADS + h].set(adst[h])
        return a

    a0 = amat(gat0_asrc, gat0_adst)
    a1 = amat(gat1_asrc, gat1_adst)
    b0flat = gat0_b.reshape(1, -1)
    b1flat = gat1_b.reshape(1, -1)
    z = jnp.zeros((NPAD, WX), jnp.float32)

    grid = (N // BLK,)
    hhx0, sd0 = pl.pallas_call(
        _dense1_body,
        grid=grid,
        in_specs=[
            _rows(NODE), _rows(NODE), _rows(EDGE), _rows(1), _rows(1),
            _full((1, TIME)), _full((1, TIME)),
            _full((NODE, MSG)), _full((NODE, MSG)), _full((EDGE, MSG)),
            _full((TIME, MSG)), _full((1, MSG)), _full((MSG, MSG)),
            _full((1, MSG)), _full((MSG, 3 * MEM)), _full((1, 3 * MEM)),
            _full((1, 3 * MEM)), _full((L0_IN, HEADS * HPD)),
            _full((HEADS * HPD, 2 * HEADS)),
        ],
        out_specs=[_rows(WX), _rows(16)],
        out_shape=[
            jax.ShapeDtypeStruct((N, WX), jnp.float32),
            jax.ShapeDtypeStruct((N, 16), jnp.float32),
        ],
    )(node_features, nfd2p, eap, ttp[:, None], has_f,
      time_w[None, :], time_b[None, :], w1a, w1c, w1d, w1e,
      msg_b1[None, :], msg_W2, msg_b2[None, :],
      wiht, gru_bih[None, :], gru_bhh[None, :], w0cat, a0)

    acc0 = _gat_aggregate(src, dst, hhx0, sd0, z)

    hhx1, sd1 = pl.pallas_call(
        _finish_proj_body,
        grid=grid,
        in_specs=[_rows(WX), _rows(WX), _full((1, HEADS * HPD)),
                  _full((NODE, HEADS * HPD)), _full((HEADS * HPD, 2 * HEADS))],
        out_specs=[_rows(WX), _rows(16)],
        out_shape=[
            jax.ShapeDtypeStruct((N, WX), jnp.float32),
            jax.ShapeDtypeStruct((N, 16), jnp.float32),
        ],
    )(acc0[0], acc0[1], b0flat, w1cat, a1)

    acc1 = _gat_aggregate(src, dst, hhx1, sd1, z)

    logits = pl.pallas_call(
        _finish_cls_body,
        grid=grid,
        in_specs=[_rows(WX), _rows(WX), _full((1, HEADS * HPD)),
                  _full((NODE, NODE // 2)), _full((1, NODE // 2)),
                  _full((NODE // 2, 1)), _full((1, 1))],
        out_specs=_rows(1),
        out_shape=jax.ShapeDtypeStruct((N, 1), jnp.float32),
    )(acc1[0], acc1[1], b1flat,
      cls_W1, cls_b1[None, :], cls_W2, cls_b2[None, :])

    return logits


# speedup vs baseline: 1.7110x; 1.2027x over previous
import functools

import jax
import jax.numpy as jnp
from jax import lax
from jax.experimental import pallas as pl
from jax.experimental.pallas import tpu as pltpu
from jax.experimental.pallas import tpu_sc as plsc

N = 10000
E = 320000
NODE = 128
EDGE = 16
TIME = 32
MEM = 128
HEADS = 4
HPD = 32
MSG = NODE + EDGE + TIME
L0_IN = NODE + MEM + TIME

BLK = 1000
NW = 32
EPW = E // NW
KW = 80
NWIN = EPW // KW
NPAD = 10240
NPS = NPAD // 16
WX = 144
SS0 = NODE
DN0 = NODE + HEADS


def _pack_ext(hh, s):
    b = hh.shape[0]
    return jnp.concatenate(
        [hh, s[:, :HEADS], jnp.ones((b, HEADS), jnp.float32),
         jnp.zeros((b, WX - DN0 - HEADS), jnp.float32)], axis=1)


def _pack_sd(s):
    b = s.shape[0]
    return jnp.concatenate(
        [s[:, HEADS:], jnp.zeros((b, 12), jnp.float32)], axis=1)


def _dense1_body(nf_ref, nfd2_ref, ea_ref, tt_ref, has_ref, tw_ref, tb_ref,
                 w1a_ref, w1c_ref, w1d_ref, w1e_ref, b1_ref, w2_ref, b2_ref,
                 wiht_ref, bih_ref, bhh_ref, w0_ref, a0_ref,
                 hhx_ref, sd_ref):
    nf = nf_ref[...]
    t = tt_ref[...] * has_ref[...]
    te = jnp.sin(tw_ref[...] * t + tb_ref[...])
    u = (nf @ w1a_ref[...] + nfd2_ref[...] @ w1c_ref[...]
         + ea_ref[...] @ w1d_ref[...] + te @ w1e_ref[...] + b1_ref[...])
    u = jnp.maximum(u, 0.0)
    msgs = u @ w2_ref[...] + b2_ref[...]
    gi = msgs @ wiht_ref[...] + bih_ref[...]
    gh = bhh_ref[...]
    ir, iz, inn = gi[:, :MEM], gi[:, MEM:2 * MEM], gi[:, 2 * MEM:]
    hr, hz, hn = gh[:, :MEM], gh[:, MEM:2 * MEM], gh[:, 2 * MEM:]
    r = jax.nn.sigmoid(ir + hr)
    z = jax.nn.sigmoid(iz + hz)
    nn_ = jnp.tanh(inn + r * hn)
    newh = (1.0 - z) * nn_
    memv = newh * has_ref[...]
    x = jnp.concatenate([nf, memv, te], axis=1)
    hh0 = x @ w0_ref[...]
    s0 = hh0 @ a0_ref[...]
    hhx_ref[...] = _pack_ext(hh0, s0)
    sd_ref[...] = _pack_sd(s0)


def _unpack_finish(a0_ref, a1_ref, b_ref):
    acc = a0_ref[...] + a1_ref[...]
    parts = []
    for h in range(HEADS):
        parts.append(acc[:, h * HPD:(h + 1) * HPD]
                     / (acc[:, DN0 + h:DN0 + h + 1] + 1e-16))
    return jnp.concatenate(parts, axis=1) + b_ref[...]


def _finish_proj_body(a0_ref, a1_ref, b_ref, w_ref, am_ref, hhx_ref, sd_ref):
    x = _unpack_finish(a0_ref, a1_ref, b_ref)
    hh = x @ w_ref[...]
    s = hh @ am_ref[...]
    hhx_ref[...] = _pack_ext(hh, s)
    sd_ref[...] = _pack_sd(s)


def _finish_cls_body(a0_ref, a1_ref, b_ref, w1_ref, b1_ref, w2_ref, b2_ref,
                     out_ref):
    x = _unpack_finish(a0_ref, a1_ref, b_ref)
    u = jnp.maximum(x @ w1_ref[...] + b1_ref[...], 0.0)
    out_ref[...] = u @ w2_ref[...] + b2_ref[...]


def _full(shape):
    return pl.BlockSpec(shape, lambda i: (0,) * len(shape))


def _rows(width):
    return pl.BlockSpec((BLK, width), lambda i: (i, 0))


def _bcast_lane(vec, j):
    return lax.gather(
        vec, jnp.full((16, 1), j, jnp.int32),
        lax.GatherDimensionNumbers(offset_dims=(), collapsed_slice_dims=(0,),
                                   start_index_map=(0,)),
        (1,), mode=lax.GatherScatterMode.PROMISE_IN_BOUNDS)


def _shift4(vec):
    idx = jnp.maximum(lax.iota(jnp.int32, 16) - 4, 0)
    return lax.gather(
        vec, idx[:, None],
        lax.GatherDimensionNumbers(offset_dims=(), collapsed_slice_dims=(0,),
                                   start_index_map=(0,)),
        (1,), mode=lax.GatherScatterMode.PROMISE_IN_BOUNDS)


def _gat_sc_body(src_ref, dst_ref, hhx_ref, sd_ref, z_ref,
                 acc_ref,
                 idx_s, idx_d, rows, sdb, out_sp,
                 sg0, sg1, ss0, ss1):
    cid = lax.axis_index("c")
    sid = lax.axis_index("s")
    wid = sid * 2 + cid
    sg = (sg0, sg1)
    ss = (ss0, ss1)

    pltpu.sync_copy(z_ref.at[pl.ds(sid * NPS, NPS)],
                    out_sp.at[pl.ds(sid * NPS, NPS)])
    plsc.subcore_barrier()

    base_edge = wid * EPW

    def load_idx(w, b):
        wb = base_edge + w * KW
        pltpu.sync_copy(src_ref.at[pl.ds(wb, KW)], idx_s.at[b])
        pltpu.sync_copy(dst_ref.at[pl.ds(wb, KW)], idx_d.at[b])

    def start_gather(b):
        pltpu.async_copy(hhx_ref.at[idx_s.at[b]], rows.at[b], sg[b])
        pltpu.async_copy(sd_ref.at[idx_d.at[b]], sdb.at[b], sg[b])

    def wait_gather(b):
        pltpu.make_async_copy(hhx_ref.at[idx_s.at[b]], rows.at[b], sg[b]).wait()
        pltpu.make_async_copy(sd_ref.at[idx_d.at[b]], sdb.at[b], sg[b]).wait()

    def start_scatter(b):
        pltpu.async_copy(rows.at[b], out_sp.at[idx_d.at[b]], ss[b], add=True)

    def wait_scatter(b):
        pltpu.make_async_copy(rows.at[b], out_sp.at[idx_d.at[b]], ss[b]).wait()

    def compute(b):
        for j in range(KW):
            sseg = rows[b, j, pl.ds(SS0, 16)]
            v = sseg + sdb[b, j, :]
            v = jnp.where(v >= 0.0, v, 0.2 * v)
            w16 = jnp.exp(v)
            rows[b, j, pl.ds(SS0, 16)] = sseg * _shift4(w16)
            for h in range(HEADS):
                wh = _bcast_lane(w16, h)
                c0 = h * HPD
                rows[b, j, pl.ds(c0, 16)] = rows[b, j, pl.ds(c0, 16)] * wh
                rows[b, j, pl.ds(c0 + 16, 16)] = (
                    rows[b, j, pl.ds(c0 + 16, 16)] * wh)

    load_idx(0, 0)
    start_gather(0)

    def step(i, carry):
        @pl.when(i > 0)
        def _():
            wait_scatter(1)
        load_idx(2 * i + 1, 1)
        start_gather(1)
        wait_gather(0)
        compute(0)
        start_scatter(0)
        load_idx(2 * i + 2, 0)
        wait_scatter(0)
        start_gather(0)
        wait_gather(1)
        compute(1)
        start_scatter(1)
        return carry

    lax.fori_loop(0, (NWIN - 1) // 2, step, 0)
    wait_scatter(1)
    wait_gather(0)
    compute(0)
    start_scatter(0)
    wait_scatter(0)
    plsc.subcore_barrier()
    pltpu.sync_copy(out_sp.at[pl.ds(sid * NPS, NPS)],
                    acc_ref.at[cid, pl.ds(sid * NPS, NPS)])


def _gat_aggregate(src, dst, hhx, sd, z):
    mesh = plsc.VectorSubcoreMesh(core_axis_name="c", subcore_axis_name="s")
    f = pl.kernel(
        _gat_sc_body,
        mesh=mesh,
        compiler_params=pltpu.CompilerParams(use_tc_tiling_on_sc=False,
                                             needs_layout_passes=False),
        out_type=jax.ShapeDtypeStruct((2, NPAD, WX), jnp.float32),
        scratch_types=[
            pltpu.VMEM((2, KW), jnp.int32),
            pltpu.VMEM((2, KW), jnp.int32),
            pltpu.VMEM((2, KW, WX), jnp.float32),
            pltpu.VMEM((2, KW, 16), jnp.float32),
            pltpu.VMEM_SHARED((NPAD, WX), jnp.float32),
            pltpu.SemaphoreType.DMA,
            pltpu.SemaphoreType.DMA,
            pltpu.SemaphoreType.DMA,
            pltpu.SemaphoreType.DMA,
        ],
    )
    return f(src, dst, hhx, sd, z)


def _shift_up1(vec):
    idx = jnp.minimum(lax.iota(jnp.int32, 16) + 1, 15)
    return lax.gather(
        vec, idx[:, None],
        lax.GatherDimensionNumbers(offset_dims=(), collapsed_slice_dims=(0,),
                                   start_index_map=(0,)),
        (1,), mode=lax.GatherScatterMode.PROMISE_IN_BOUNDS)


EPT = E // 16
SCH = 4000
NSL = NPAD // 16
NCH = NSL // 128


def _last_sc_body(src_ref, dst_ref, et_ref, ea_ref, nf_ref,
                  le_ref, tt_ref, eag_ref, nfd2_ref,
                  lastl, sbuf, mrg, leraw, lec, d2b, ttb, eab, nfb,
                  shared, sem):
    cid = lax.axis_index("c")
    sid = lax.axis_index("s")
    iota = lax.iota(jnp.int32, 16)

    @pl.when(cid == 0)
    def _():
        def initb(i, c):
            lastl[pl.ds(i * 16, 16)] = jnp.full((16,), -1, jnp.int32)
            return c
        lax.fori_loop(0, NPAD // 16, initb, 0)

        base = sid * EPT

        def chunk(ci, c0):
            pltpu.sync_copy(src_ref.at[pl.ds(base + ci * SCH, SCH)], sbuf)

            def scan(i, c):
                sv = sbuf[pl.ds(i * 16, 16)]
                key = sv * 16 + iota
                eidv = jnp.full((16,), base + ci * SCH, jnp.int32) + i * 16 + iota
                ks, vs = plsc.sort_key_val(key, eidv)
                srcs = lax.shift_right_logical(ks, 4)
                nxt = _shift_up1(srcs)
                msk = jnp.logical_or(srcs != nxt, iota == 15)
                plsc.store_scatter(lastl, [srcs], vs, mask=msk)
                return c
            lax.fori_loop(0, SCH // 16, scan, 0)
            return c0
        lax.fori_loop(0, EPT // SCH, chunk, 0)

        pltpu.sync_copy(lastl, shared.at[sid])
        plsc.subcore_barrier()
        for k in range(NCH):
            pltpu.sync_copy(shared.at[:, pl.ds(sid * NSL + k * 128, 128)], mrg)

            def red(q, c):
                acc = mrg[0, pl.ds(q * 16, 16)]
                for r in range(1, 16):
                    acc = jnp.maximum(acc, mrg[r, pl.ds(q * 16, 16)])
                leraw[pl.ds(k * 128 + q * 16, 16)] = acc
                lec[k, pl.ds(q * 16, 16)] = jnp.maximum(acc, 0)
                return c
            lax.fori_loop(0, 8, red, 0)

        pltpu.sync_copy(leraw, le_ref.at[pl.ds(sid * NSL, NSL)])
        cps = [pltpu.async_copy(dst_ref.at[lec.at[k]], d2b.at[k], sem)
               for k in range(NCH)]
        for cp in cps:
            cp.wait()
        cps = []
        for k in range(NCH):
            cps.append(pltpu.async_copy(et_ref.at[lec.at[k]], ttb.at[k], sem))
            cps.append(pltpu.async_copy(
                ea_ref.at[lec.at[k]], eab.at[pl.ds(k * 128, 128)], sem))
            cps.append(pltpu.async_copy(
                nf_ref.at[d2b.at[k]], nfb.at[pl.ds(k * 128, 128)], sem))
        for cp in cps:
            cp.wait()
        for k in range(NCH):
            pltpu.sync_copy(ttb.at[k],
                            tt_ref.at[pl.ds(sid * NSL + k * 128, 128)])
        pltpu.sync_copy(eab, eag_ref.at[pl.ds(sid * NSL, NSL)])
        pltpu.sync_copy(nfb, nfd2_ref.at[pl.ds(sid * NSL, NSL)])


def _last_edge(src, dst, et, ea, nf):
    mesh = plsc.VectorSubcoreMesh(core_axis_name="c", subcore_axis_name="s")
    f = pl.kernel(
        _last_sc_body,
        mesh=mesh,
        compiler_params=pltpu.CompilerParams(use_tc_tiling_on_sc=False,
                                             needs_layout_passes=False),
        out_type=[
            jax.ShapeDtypeStruct((NPAD,), jnp.int32),
            jax.ShapeDtypeStruct((NPAD,), jnp.float32),
            jax.ShapeDtypeStruct((NPAD, EDGE), jnp.float32),
            jax.ShapeDtypeStruct((NPAD, NODE), jnp.float32),
        ],
        scratch_types=[
            pltpu.VMEM((NPAD,), jnp.int32),
            pltpu.VMEM((SCH,), jnp.int32),
            pltpu.VMEM((16, 128), jnp.int32),
            pltpu.VMEM((NSL,), jnp.int32),
            pltpu.VMEM((NCH, 128), jnp.int32),
            pltpu.VMEM((NCH, 128), jnp.int32),
            pltpu.VMEM((NCH, 128), jnp.float32),
            pltpu.VMEM((NSL, EDGE), jnp.float32),
            pltpu.VMEM((NSL, NODE), jnp.float32),
            pltpu.VMEM_SHARED((16, NPAD), jnp.int32),
            pltpu.SemaphoreType.DMA,
        ],
    )
    return f(src, dst, et, ea, nf)


def kernel(node_features, edge_index, edge_attr, edge_times, time_w, time_b,
           msg_W1, msg_b1, msg_W2, msg_b2, gru_Wih, gru_Whh, gru_bih, gru_bhh,
           gat0_W, gat0_asrc, gat0_adst, gat0_b,
           gat1_W, gat1_asrc, gat1_adst, gat1_b,
           cls_W1, cls_b1, cls_W2, cls_b2):
    src = edge_index[0]
    dst = edge_index[1]
    lep, ttp, eap, nfd2p = _last_edge(src, dst, edge_times, edge_attr,
                                      node_features)
    has_f = (lep[:N] >= 0).astype(jnp.float32)[:, None]

    w1a = msg_W1[:NODE]
    w1c = msg_W1[NODE + MEM:NODE + MEM + NODE]
    w1d = msg_W1[NODE + MEM + NODE:NODE + MEM + NODE + EDGE]
    w1e = msg_W1[NODE + MEM + NODE + EDGE:]
    wiht = gru_Wih.T
    w0cat = jnp.transpose(gat0_W, (1, 0, 2)).reshape(L0_IN, HEADS * HPD)
    w1cat = jnp.transpose(gat1_W, (1, 0, 2)).reshape(NODE, HEADS * HPD)

    def amat(asrc, adst):
        a = jnp.zeros((HEADS * HPD, 2 * HEADS), jnp.float32)
        for h in range(HEADS):
            a = a.at[h * HPD:(h + 1) * HPD, h].set(asrc[h])
            a = a.at[h * HPD:(h + 1) * HPD, HEADS + h].set(adst[h])
        return a

    a0 = amat(gat0_asrc, gat0_adst)
    a1 = amat(gat1_asrc, gat1_adst)
    b0flat = gat0_b.reshape(1, -1)
    b1flat = gat1_b.reshape(1, -1)
    z = jnp.zeros((NPAD, WX), jnp.float32)

    grid = (N // BLK,)
    hhx0, sd0 = pl.pallas_call(
        _dense1_body,
        grid=grid,
        in_specs=[
            _rows(NODE), _rows(NODE), _rows(EDGE), _rows(1), _rows(1),
            _full((1, TIME)), _full((1, TIME)),
            _full((NODE, MSG)), _full((NODE, MSG)), _full((EDGE, MSG)),
            _full((TIME, MSG)), _full((1, MSG)), _full((MSG, MSG)),
            _full((1, MSG)), _full((MSG, 3 * MEM)), _full((1, 3 * MEM)),
            _full((1, 3 * MEM)), _full((L0_IN, HEADS * HPD)),
            _full((HEADS * HPD, 2 * HEADS)),
        ],
        out_specs=[_rows(WX), _rows(16)],
        out_shape=[
            jax.ShapeDtypeStruct((N, WX), jnp.float32),
            jax.ShapeDtypeStruct((N, 16), jnp.float32),
        ],
    )(node_features, nfd2p, eap, ttp[:, None], has_f,
      time_w[None, :], time_b[None, :], w1a, w1c, w1d, w1e,
      msg_b1[None, :], msg_W2, msg_b2[None, :],
      wiht, gru_bih[None, :], gru_bhh[None, :], w0cat, a0)

    acc0 = _gat_aggregate(src, dst, hhx0, sd0, z)

    hhx1, sd1 = pl.pallas_call(
        _finish_proj_body,
        grid=grid,
        in_specs=[_rows(WX), _rows(WX), _full((1, HEADS * HPD)),
                  _full((NODE, HEADS * HPD)), _full((HEADS * HPD, 2 * HEADS))],
        out_specs=[_rows(WX), _rows(16)],
        out_shape=[
            jax.ShapeDtypeStruct((N, WX), jnp.float32),
            jax.ShapeDtypeStruct((N, 16), jnp.float32),
        ],
    )(acc0[0], acc0[1], b0flat, w1cat, a1)

    acc1 = _gat_aggregate(src, dst, hhx1, sd1, z)

    logits = pl.pallas_call(
        _finish_cls_body,
        grid=grid,
        in_specs=[_rows(WX), _rows(WX), _full((1, HEADS * HPD)),
                  _full((NODE, NODE // 2)), _full((1, NODE // 2)),
                  _full((NODE // 2, 1)), _full((1, 1))],
        out_specs=_rows(1),
        out_shape=jax.ShapeDtypeStruct((N, 1), jnp.float32),
    )(acc1[0], acc1[1], b1flat,
      cls_W1, cls_b1[None, :], cls_W2, cls_b2[None, :])

    return logits


# speedup vs baseline: 1.7596x; 1.0284x over previous
import functools

import jax
import jax.numpy as jnp
from jax import lax
from jax.experimental import pallas as pl
from jax.experimental.pallas import tpu as pltpu
from jax.experimental.pallas import tpu_sc as plsc

N = 10000
E = 320000
NODE = 128
EDGE = 16
TIME = 32
MEM = 128
HEADS = 4
HPD = 32
MSG = NODE + EDGE + TIME
L0_IN = NODE + MEM + TIME

BLK = 1000
NW = 32
EPW = E // NW
KW = 80
NWIN = EPW // KW
NPAD = 10240
NPS = NPAD // 16
WX = 144
SS0 = NODE
DN0 = NODE + HEADS


def _pack_ext(hh, s):
    b = hh.shape[0]
    return jnp.concatenate(
        [hh, s[:, :HEADS], jnp.ones((b, HEADS), jnp.float32),
         jnp.zeros((b, WX - DN0 - HEADS), jnp.float32)], axis=1)


def _pack_sd(s):
    b = s.shape[0]
    return jnp.concatenate(
        [s[:, HEADS:], jnp.zeros((b, 12), jnp.float32)], axis=1)


def _dense1_body(nf_ref, nfd2_ref, ea_ref, tt_ref, has_ref, tw_ref, tb_ref,
                 w1a_ref, w1c_ref, w1d_ref, w1e_ref, b1_ref, w2_ref, b2_ref,
                 wiht_ref, bih_ref, bhh_ref, w0_ref, a0_ref,
                 hhx_ref, sd_ref):
    nf = nf_ref[...]
    t = tt_ref[...] * has_ref[...]
    te = jnp.sin(tw_ref[...] * t + tb_ref[...])
    u = (nf @ w1a_ref[...] + nfd2_ref[...] @ w1c_ref[...]
         + ea_ref[...] @ w1d_ref[...] + te @ w1e_ref[...] + b1_ref[...])
    u = jnp.maximum(u, 0.0)
    msgs = u @ w2_ref[...] + b2_ref[...]
    gi = msgs @ wiht_ref[...] + bih_ref[...]
    gh = bhh_ref[...]
    ir, iz, inn = gi[:, :MEM], gi[:, MEM:2 * MEM], gi[:, 2 * MEM:]
    hr, hz, hn = gh[:, :MEM], gh[:, MEM:2 * MEM], gh[:, 2 * MEM:]
    r = jax.nn.sigmoid(ir + hr)
    z = jax.nn.sigmoid(iz + hz)
    nn_ = jnp.tanh(inn + r * hn)
    newh = (1.0 - z) * nn_
    memv = newh * has_ref[...]
    x = jnp.concatenate([nf, memv, te], axis=1)
    hh0 = x @ w0_ref[...]
    s0 = hh0 @ a0_ref[...]
    hhx_ref[...] = _pack_ext(hh0, s0)
    sd_ref[...] = _pack_sd(s0)


def _unpack_finish(a0_ref, a1_ref, b_ref):
    acc = a0_ref[0] + a1_ref[0]
    parts = []
    for h in range(HEADS):
        parts.append(acc[:, h * HPD:(h + 1) * HPD]
                     / (acc[:, DN0 + h:DN0 + h + 1] + 1e-16))
    return jnp.concatenate(parts, axis=1) + b_ref[...]


def _finish_proj_body(a0_ref, a1_ref, b_ref, w_ref, am_ref, hhx_ref, sd_ref):
    x = _unpack_finish(a0_ref, a1_ref, b_ref)
    hh = x @ w_ref[...]
    s = hh @ am_ref[...]
    hhx_ref[...] = _pack_ext(hh, s)
    sd_ref[...] = _pack_sd(s)


def _finish_cls_body(a0_ref, a1_ref, b_ref, w1_ref, b1_ref, w2_ref, b2_ref,
                     out_ref):
    x = _unpack_finish(a0_ref, a1_ref, b_ref)
    u = jnp.maximum(x @ w1_ref[...] + b1_ref[...], 0.0)
    out_ref[...] = u @ w2_ref[...] + b2_ref[...]


def _full(shape):
    return pl.BlockSpec(shape, lambda i: (0,) * len(shape))


def _rows(width):
    return pl.BlockSpec((BLK, width), lambda i: (i, 0))


def _bcast_lane(vec, j):
    return lax.gather(
        vec, jnp.full((16, 1), j, jnp.int32),
        lax.GatherDimensionNumbers(offset_dims=(), collapsed_slice_dims=(0,),
                                   start_index_map=(0,)),
        (1,), mode=lax.GatherScatterMode.PROMISE_IN_BOUNDS)


def _shift4(vec):
    idx = jnp.maximum(lax.iota(jnp.int32, 16) - 4, 0)
    return lax.gather(
        vec, idx[:, None],
        lax.GatherDimensionNumbers(offset_dims=(), collapsed_slice_dims=(0,),
                                   start_index_map=(0,)),
        (1,), mode=lax.GatherScatterMode.PROMISE_IN_BOUNDS)


def _gat_sc_body(src_ref, dst_ref, hhx_ref, sd_ref, z_ref,
                 acc_ref,
                 idx_s, idx_d, rows, sdb, out_sp,
                 sg0, sg1, ss0, ss1):
    cid = lax.axis_index("c")
    sid = lax.axis_index("s")
    wid = sid * 2 + cid
    sg = (sg0, sg1)
    ss = (ss0, ss1)

    pltpu.sync_copy(z_ref.at[pl.ds(sid * NPS, NPS)],
                    out_sp.at[pl.ds(sid * NPS, NPS)])
    plsc.subcore_barrier()

    base_edge = wid * EPW

    def load_idx(w, b):
        wb = base_edge + w * KW
        pltpu.sync_copy(src_ref.at[pl.ds(wb, KW)], idx_s.at[b])
        pltpu.sync_copy(dst_ref.at[pl.ds(wb, KW)], idx_d.at[b])

    def start_gather(b):
        pltpu.async_copy(hhx_ref.at[idx_s.at[b]], rows.at[b], sg[b])
        pltpu.async_copy(sd_ref.at[idx_d.at[b]], sdb.at[b], sg[b])

    def wait_gather(b):
        pltpu.make_async_copy(hhx_ref.at[idx_s.at[b]], rows.at[b], sg[b]).wait()
        pltpu.make_async_copy(sd_ref.at[idx_d.at[b]], sdb.at[b], sg[b]).wait()

    def start_scatter(b):
        pltpu.async_copy(rows.at[b], out_sp.at[idx_d.at[b]], ss[b], add=True)

    def wait_scatter(b):
        pltpu.make_async_copy(rows.at[b], out_sp.at[idx_d.at[b]], ss[b]).wait()

    def compute(b):
        for j in range(KW):
            sseg = rows[b, j, pl.ds(SS0, 16)]
            v = sseg + sdb[b, j, :]
            v = jnp.where(v >= 0.0, v, 0.2 * v)
            w16 = jnp.exp(v)
            rows[b, j, pl.ds(SS0, 16)] = sseg * _shift4(w16)
            for h in range(HEADS):
                wh = _bcast_lane(w16, h)
                c0 = h * HPD
                rows[b, j, pl.ds(c0, 16)] = rows[b, j, pl.ds(c0, 16)] * wh
                rows[b, j, pl.ds(c0 + 16, 16)] = (
                    rows[b, j, pl.ds(c0 + 16, 16)] * wh)

    load_idx(0, 0)
    start_gather(0)

    def step(i, carry):
        @pl.when(i > 0)
        def _():
            wait_scatter(1)
        load_idx(2 * i + 1, 1)
        start_gather(1)
        wait_gather(0)
        compute(0)
        start_scatter(0)
        load_idx(2 * i + 2, 0)
        wait_scatter(0)
        start_gather(0)
        wait_gather(1)
        compute(1)
        start_scatter(1)
        return carry

    lax.fori_loop(0, (NWIN - 1) // 2, step, 0)
    wait_scatter(1)
    wait_gather(0)
    compute(0)
    start_scatter(0)
    wait_scatter(0)
    plsc.subcore_barrier()
    pltpu.sync_copy(out_sp.at[pl.ds(sid * NPS, NPS)],
                    acc_ref.at[cid, pl.ds(sid * NPS, NPS)])


def _gat_aggregate(src, dst, hhx, sd, z):
    mesh = plsc.VectorSubcoreMesh(core_axis_name="c", subcore_axis_name="s")
    f = pl.kernel(
        _gat_sc_body,
        mesh=mesh,
        compiler_params=pltpu.CompilerParams(use_tc_tiling_on_sc=False,
                                             needs_layout_passes=False),
        out_type=jax.ShapeDtypeStruct((2, NPAD, WX), jnp.float32),
        scratch_types=[
            pltpu.VMEM((2, KW), jnp.int32),
            pltpu.VMEM((2, KW), jnp.int32),
            pltpu.VMEM((2, KW, WX), jnp.float32),
            pltpu.VMEM((2, KW, 16), jnp.float32),
            pltpu.VMEM_SHARED((NPAD, WX), jnp.float32),
            pltpu.SemaphoreType.DMA,
            pltpu.SemaphoreType.DMA,
            pltpu.SemaphoreType.DMA,
            pltpu.SemaphoreType.DMA,
        ],
    )
    return f(src, dst, hhx, sd, z)


def _shift_up1(vec):
    idx = jnp.minimum(lax.iota(jnp.int32, 16) + 1, 15)
    return lax.gather(
        vec, idx[:, None],
        lax.GatherDimensionNumbers(offset_dims=(), collapsed_slice_dims=(0,),
                                   start_index_map=(0,)),
        (1,), mode=lax.GatherScatterMode.PROMISE_IN_BOUNDS)


EPT = E // 16
SCH = 4000
NSL = NPAD // 16
NCH = NSL // 128


def _last_sc_body(src_ref, dst_ref, et_ref, ea_ref, nf_ref,
                  le_ref, tt_ref, eag_ref, nfd2_ref,
                  lastl, sbuf, mrg, leraw, lec, d2b, ttb, eab, nfb,
                  shared, sem):
    cid = lax.axis_index("c")
    sid = lax.axis_index("s")
    iota = lax.iota(jnp.int32, 16)

    @pl.when(cid == 0)
    def _():
        def initb(i, c):
            lastl[pl.ds(i * 16, 16)] = jnp.full((16,), -1, jnp.int32)
            return c
        lax.fori_loop(0, NPAD // 16, initb, 0)

        base = sid * EPT

        def chunk(ci, c0):
            pltpu.sync_copy(src_ref.at[pl.ds(base + ci * SCH, SCH)], sbuf)

            def scan(i, c):
                sv = sbuf[pl.ds(i * 16, 16)]
                key = sv * 16 + iota
                eidv = jnp.full((16,), base + ci * SCH, jnp.int32) + i * 16 + iota
                ks, vs = plsc.sort_key_val(key, eidv)
                srcs = lax.shift_right_logical(ks, 4)
                nxt = _shift_up1(srcs)
                msk = jnp.logical_or(srcs != nxt, iota == 15)
                plsc.store_scatter(lastl, [srcs], vs, mask=msk)
                return c
            lax.fori_loop(0, SCH // 16, scan, 0)
            return c0
        lax.fori_loop(0, EPT // SCH, chunk, 0)

        pltpu.sync_copy(lastl, shared.at[sid])
        plsc.subcore_barrier()
        for k in range(NCH):
            pltpu.sync_copy(shared.at[:, pl.ds(sid * NSL + k * 128, 128)], mrg)

            def red(q, c):
                acc = mrg[0, pl.ds(q * 16, 16)]
                for r in range(1, 16):
                    acc = jnp.maximum(acc, mrg[r, pl.ds(q * 16, 16)])
                leraw[pl.ds(k * 128 + q * 16, 16)] = acc
                lec[k, pl.ds(q * 16, 16)] = jnp.maximum(acc, 0)
                return c
            lax.fori_loop(0, 8, red, 0)

        pltpu.sync_copy(leraw, le_ref.at[pl.ds(sid * NSL, NSL)])
        cps = [pltpu.async_copy(dst_ref.at[lec.at[k]], d2b.at[k], sem)
               for k in range(NCH)]
        for cp in cps:
            cp.wait()
        cps = []
        for k in range(NCH):
            cps.append(pltpu.async_copy(et_ref.at[lec.at[k]], ttb.at[k], sem))
            cps.append(pltpu.async_copy(
                ea_ref.at[lec.at[k]], eab.at[pl.ds(k * 128, 128)], sem))
            cps.append(pltpu.async_copy(
                nf_ref.at[d2b.at[k]], nfb.at[pl.ds(k * 128, 128)], sem))
        for cp in cps:
            cp.wait()
        for k in range(NCH):
            pltpu.sync_copy(ttb.at[k],
                            tt_ref.at[pl.ds(sid * NSL + k * 128, 128)])
        pltpu.sync_copy(eab, eag_ref.at[pl.ds(sid * NSL, NSL)])
        pltpu.sync_copy(nfb, nfd2_ref.at[pl.ds(sid * NSL, NSL)])


def _last_edge(src, dst, et, ea, nf):
    mesh = plsc.VectorSubcoreMesh(core_axis_name="c", subcore_axis_name="s")
    f = pl.kernel(
        _last_sc_body,
        mesh=mesh,
        compiler_params=pltpu.CompilerParams(use_tc_tiling_on_sc=False,
                                             needs_layout_passes=False),
        out_type=[
            jax.ShapeDtypeStruct((NPAD,), jnp.int32),
            jax.ShapeDtypeStruct((NPAD,), jnp.float32),
            jax.ShapeDtypeStruct((NPAD, EDGE), jnp.float32),
            jax.ShapeDtypeStruct((NPAD, NODE), jnp.float32),
        ],
        scratch_types=[
            pltpu.VMEM((NPAD,), jnp.int32),
            pltpu.VMEM((SCH,), jnp.int32),
            pltpu.VMEM((16, 128), jnp.int32),
            pltpu.VMEM((NSL,), jnp.int32),
            pltpu.VMEM((NCH, 128), jnp.int32),
            pltpu.VMEM((NCH, 128), jnp.int32),
            pltpu.VMEM((NCH, 128), jnp.float32),
            pltpu.VMEM((NSL, EDGE), jnp.float32),
            pltpu.VMEM((NSL, NODE), jnp.float32),
            pltpu.VMEM_SHARED((16, NPAD), jnp.int32),
            pltpu.SemaphoreType.DMA,
        ],
    )
    return f(src, dst, et, ea, nf)


def kernel(node_features, edge_index, edge_attr, edge_times, time_w, time_b,
           msg_W1, msg_b1, msg_W2, msg_b2, gru_Wih, gru_Whh, gru_bih, gru_bhh,
           gat0_W, gat0_asrc, gat0_adst, gat0_b,
           gat1_W, gat1_asrc, gat1_adst, gat1_b,
           cls_W1, cls_b1, cls_W2, cls_b2):
    src = edge_index[0]
    dst = edge_index[1]
    lep, ttp, eap, nfd2p = _last_edge(src, dst, edge_times, edge_attr,
                                      node_features)
    has_f = (lep[:N] >= 0).astype(jnp.float32)[:, None]

    w1a = msg_W1[:NODE]
    w1c = msg_W1[NODE + MEM:NODE + MEM + NODE]
    w1d = msg_W1[NODE + MEM + NODE:NODE + MEM + NODE + EDGE]
    w1e = msg_W1[NODE + MEM + NODE + EDGE:]
    wiht = gru_Wih.T
    w0cat = jnp.transpose(gat0_W, (1, 0, 2)).reshape(L0_IN, HEADS * HPD)
    w1cat = jnp.transpose(gat1_W, (1, 0, 2)).reshape(NODE, HEADS * HPD)

    def amat(asrc, adst):
        a = jnp.zeros((HEADS * HPD, 2 * HEADS), jnp.float32)
        for h in range(HEADS):
            a = a.at[h * HPD:(h + 1) * HPD, h].set(asrc[h])
            a = a.at[h * HPD:(h + 1) * HPD, HEADS + h].set(adst[h])
        return a

    a0 = amat(gat0_asrc, gat0_adst)
    a1 = amat(gat1_asrc, gat1_adst)
    b0flat = gat0_b.reshape(1, -1)
    b1flat = gat1_b.reshape(1, -1)
    z = jnp.zeros((NPAD, WX), jnp.float32)

    grid = (N // BLK,)
    hhx0, sd0 = pl.pallas_call(
        _dense1_body,
        grid=grid,
        in_specs=[
            _rows(NODE), _rows(NODE), _rows(EDGE), _rows(1), _rows(1),
            _full((1, TIME)), _full((1, TIME)),
            _full((NODE, MSG)), _full((NODE, MSG)), _full((EDGE, MSG)),
            _full((TIME, MSG)), _full((1, MSG)), _full((MSG, MSG)),
            _full((1, MSG)), _full((MSG, 3 * MEM)), _full((1, 3 * MEM)),
            _full((1, 3 * MEM)), _full((L0_IN, HEADS * HPD)),
            _full((HEADS * HPD, 2 * HEADS)),
        ],
        out_specs=[_rows(WX), _rows(16)],
        out_shape=[
            jax.ShapeDtypeStruct((N, WX), jnp.float32),
            jax.ShapeDtypeStruct((N, 16), jnp.float32),
        ],
    )(node_features, nfd2p, eap, ttp[:, None], has_f,
      time_w[None, :], time_b[None, :], w1a, w1c, w1d, w1e,
      msg_b1[None, :], msg_W2, msg_b2[None, :],
      wiht, gru_bih[None, :], gru_bhh[None, :], w0cat, a0)

    acc0 = _gat_aggregate(src, dst, hhx0, sd0, z)

    acc_a = pl.BlockSpec((1, BLK, WX), lambda i: (0, i, 0))
    acc_b = pl.BlockSpec((1, BLK, WX), lambda i: (1, i, 0))

    hhx1, sd1 = pl.pallas_call(
        _finish_proj_body,
        grid=grid,
        in_specs=[acc_a, acc_b, _full((1, HEADS * HPD)),
                  _full((NODE, HEADS * HPD)), _full((HEADS * HPD, 2 * HEADS))],
        out_specs=[_rows(WX), _rows(16)],
        out_shape=[
            jax.ShapeDtypeStruct((N, WX), jnp.float32),
            jax.ShapeDtypeStruct((N, 16), jnp.float32),
        ],
    )(acc0, acc0, b0flat, w1cat, a1)

    acc1 = _gat_aggregate(src, dst, hhx1, sd1, z)

    logits = pl.pallas_call(
        _finish_cls_body,
        grid=grid,
        in_specs=[acc_a, acc_b, _full((1, HEADS * HPD)),
                  _full((NODE, NODE // 2)), _full((1, NODE // 2)),
                  _full((NODE // 2, 1)), _full((1, 1))],
        out_specs=_rows(1),
        out_shape=jax.ShapeDtypeStruct((N, 1), jnp.float32),
    )(acc1, acc1, b1flat,
      cls_W1, cls_b1[None, :], cls_W2, cls_b2[None, :])

    return logits


# speedup vs baseline: 1.9076x; 1.0841x over previous
import functools

import jax
import jax.numpy as jnp
from jax import lax
from jax.experimental import pallas as pl
from jax.experimental.pallas import tpu as pltpu
from jax.experimental.pallas import tpu_sc as plsc

N = 10000
E = 320000
NODE = 128
EDGE = 16
TIME = 32
MEM = 128
HEADS = 4
HPD = 32
MSG = NODE + EDGE + TIME
L0_IN = NODE + MEM + TIME

BLK = 1000
NW = 32
EPW = E // NW
KW = 80
NWIN = EPW // KW
NPAD = 10240
NPS = NPAD // 16
WX = 144
SS0 = NODE
DN0 = NODE + HEADS


def _pack_ext(hh, s):
    b = hh.shape[0]
    return jnp.concatenate(
        [hh, s[:, :HEADS], jnp.ones((b, HEADS), jnp.float32),
         jnp.zeros((b, WX - DN0 - HEADS), jnp.float32)], axis=1)


def _pack_sd(s):
    b = s.shape[0]
    return jnp.concatenate(
        [s[:, HEADS:], jnp.zeros((b, 12), jnp.float32)], axis=1)


def _dense1_body(nf_ref, nfd2_ref, ea_ref, tt_ref, has_ref, tw_ref, tb_ref,
                 w1a_ref, w1c_ref, w1d_ref, w1e_ref, b1_ref, w2_ref, b2_ref,
                 wiht_ref, bih_ref, bhh_ref, w0_ref, a0_ref,
                 hhx_ref, sd_ref):
    nf = nf_ref[...]
    t = tt_ref[...] * has_ref[...]
    te = jnp.sin(tw_ref[...] * t + tb_ref[...])
    u = (nf @ w1a_ref[...] + nfd2_ref[...] @ w1c_ref[...]
         + ea_ref[...] @ w1d_ref[...] + te @ w1e_ref[...] + b1_ref[...])
    u = jnp.maximum(u, 0.0)
    msgs = u @ w2_ref[...] + b2_ref[...]
    gi = msgs @ wiht_ref[...] + bih_ref[...]
    gh = bhh_ref[...]
    ir, iz, inn = gi[:, :MEM], gi[:, MEM:2 * MEM], gi[:, 2 * MEM:]
    hr, hz, hn = gh[:, :MEM], gh[:, MEM:2 * MEM], gh[:, 2 * MEM:]
    r = jax.nn.sigmoid(ir + hr)
    z = jax.nn.sigmoid(iz + hz)
    nn_ = jnp.tanh(inn + r * hn)
    newh = (1.0 - z) * nn_
    memv = newh * has_ref[...]
    x = jnp.concatenate([nf, memv, te], axis=1)
    hh0 = x @ w0_ref[...]
    s0 = hh0 @ a0_ref[...]
    hhx_ref[...] = _pack_ext(hh0, s0)
    sd_ref[...] = _pack_sd(s0)


def _unpack_finish(a0_ref, a1_ref, b_ref):
    acc = a0_ref[0] + a1_ref[0]
    parts = []
    for h in range(HEADS):
        parts.append(acc[:, h * HPD:(h + 1) * HPD]
                     / (acc[:, DN0 + h:DN0 + h + 1] + 1e-16))
    return jnp.concatenate(parts, axis=1) + b_ref[...]


def _finish_proj_body(a0_ref, a1_ref, b_ref, w_ref, am_ref, hhx_ref, sd_ref):
    x = _unpack_finish(a0_ref, a1_ref, b_ref)
    hh = x @ w_ref[...]
    s = hh @ am_ref[...]
    hhx_ref[...] = _pack_ext(hh, s)
    sd_ref[...] = _pack_sd(s)


def _finish_cls_body(a0_ref, a1_ref, b_ref, w1_ref, b1_ref, w2_ref, b2_ref,
                     out_ref):
    x = _unpack_finish(a0_ref, a1_ref, b_ref)
    u = jnp.maximum(x @ w1_ref[...] + b1_ref[...], 0.0)
    out_ref[...] = u @ w2_ref[...] + b2_ref[...]


def _full(shape):
    return pl.BlockSpec(shape, lambda i: (0,) * len(shape))


def _rows(width):
    return pl.BlockSpec((BLK, width), lambda i: (i, 0))


def _bcast_lane(vec, j):
    return lax.gather(
        vec, jnp.full((16, 1), j, jnp.int32),
        lax.GatherDimensionNumbers(offset_dims=(), collapsed_slice_dims=(0,),
                                   start_index_map=(0,)),
        (1,), mode=lax.GatherScatterMode.PROMISE_IN_BOUNDS)


def _shift4(vec):
    idx = jnp.maximum(lax.iota(jnp.int32, 16) - 4, 0)
    return lax.gather(
        vec, idx[:, None],
        lax.GatherDimensionNumbers(offset_dims=(), collapsed_slice_dims=(0,),
                                   start_index_map=(0,)),
        (1,), mode=lax.GatherScatterMode.PROMISE_IN_BOUNDS)


def _gat_sc_body(src_ref, dst_ref, hhx_ref, sd_ref, z_ref,
                 acc_ref,
                 idx_s, idx_d, rows, sdb, out_sp,
                 sg0, sg1, ss0, ss1):
    cid = lax.axis_index("c")
    sid = lax.axis_index("s")
    wid = sid * 2 + cid
    sg = (sg0, sg1)
    ss = (ss0, ss1)

    pltpu.sync_copy(z_ref.at[pl.ds(sid * NPS, NPS)],
                    out_sp.at[pl.ds(sid * NPS, NPS)])
    plsc.subcore_barrier()

    base_edge = wid * EPW

    def load_idx(w, b):
        wb = base_edge + w * KW
        c1 = pltpu.async_copy(src_ref.at[pl.ds(wb, KW)], idx_s.at[b], sg[b])
        c2 = pltpu.async_copy(dst_ref.at[pl.ds(wb, KW)], idx_d.at[b], sg[b])
        c1.wait()
        c2.wait()

    def start_gather(b):
        pltpu.async_copy(hhx_ref.at[idx_s.at[b]], rows.at[b], sg[b])
        pltpu.async_copy(sd_ref.at[idx_d.at[b]], sdb.at[b], sg[b])

    def wait_gather(b):
        pltpu.make_async_copy(hhx_ref.at[idx_s.at[b]], rows.at[b], sg[b]).wait()
        pltpu.make_async_copy(sd_ref.at[idx_d.at[b]], sdb.at[b], sg[b]).wait()

    def start_scatter(b):
        pltpu.async_copy(rows.at[b], out_sp.at[idx_d.at[b]], ss[b], add=True)

    def wait_scatter(b):
        pltpu.make_async_copy(rows.at[b], out_sp.at[idx_d.at[b]], ss[b]).wait()

    def compute(b):
        for j in range(KW):
            sseg = rows[b, j, pl.ds(SS0, 16)]
            v = sseg + sdb[b, j, :]
            v = jnp.where(v >= 0.0, v, 0.2 * v)
            w16 = jnp.exp(v)
            rows[b, j, pl.ds(SS0, 16)] = sseg * _shift4(w16)
            for h in range(HEADS):
                wh = _bcast_lane(w16, h)
                c0 = h * HPD
                rows[b, j, pl.ds(c0, 16)] = rows[b, j, pl.ds(c0, 16)] * wh
                rows[b, j, pl.ds(c0 + 16, 16)] = (
                    rows[b, j, pl.ds(c0 + 16, 16)] * wh)

    load_idx(0, 0)
    start_gather(0)

    def step(i, carry):
        @pl.when(i > 0)
        def _():
            wait_scatter(1)
        load_idx(2 * i + 1, 1)
        start_gather(1)
        wait_gather(0)
        compute(0)
        start_scatter(0)
        load_idx(2 * i + 2, 0)
        wait_scatter(0)
        start_gather(0)
        wait_gather(1)
        compute(1)
        start_scatter(1)
        return carry

    lax.fori_loop(0, (NWIN - 1) // 2, step, 0)
    wait_scatter(1)
    wait_gather(0)
    compute(0)
    start_scatter(0)
    wait_scatter(0)
    plsc.subcore_barrier()
    pltpu.sync_copy(out_sp.at[pl.ds(sid * NPS, NPS)],
                    acc_ref.at[cid, pl.ds(sid * NPS, NPS)])


def _gat_aggregate(src, dst, hhx, sd, z):
    mesh = plsc.VectorSubcoreMesh(core_axis_name="c", subcore_axis_name="s")
    f = pl.kernel(
        _gat_sc_body,
        mesh=mesh,
        compiler_params=pltpu.CompilerParams(use_tc_tiling_on_sc=False,
                                             needs_layout_passes=False),
        out_type=jax.ShapeDtypeStruct((2, NPAD, WX), jnp.float32),
        scratch_types=[
            pltpu.VMEM((2, KW), jnp.int32),
            pltpu.VMEM((2, KW), jnp.int32),
            pltpu.VMEM((2, KW, WX), jnp.float32),
            pltpu.VMEM((2, KW, 16), jnp.float32),
            pltpu.VMEM_SHARED((NPAD, WX), jnp.float32),
            pltpu.SemaphoreType.DMA,
            pltpu.SemaphoreType.DMA,
            pltpu.SemaphoreType.DMA,
            pltpu.SemaphoreType.DMA,
        ],
    )
    return f(src, dst, hhx, sd, z)


def _shift_up1(vec):
    idx = jnp.minimum(lax.iota(jnp.int32, 16) + 1, 15)
    return lax.gather(
        vec, idx[:, None],
        lax.GatherDimensionNumbers(offset_dims=(), collapsed_slice_dims=(0,),
                                   start_index_map=(0,)),
        (1,), mode=lax.GatherScatterMode.PROMISE_IN_BOUNDS)


EPT = E // 16
SCH = 4000
NSL = NPAD // 16
NCH = NSL // 128


def _last_sc_body(src_ref, dst_ref, et_ref, ea_ref, nf_ref,
                  le_ref, tt_ref, eag_ref, nfd2_ref,
                  lastl, sbuf, mrg, leraw, lec, d2b, ttb, eab, nfb,
                  shared, sem):
    cid = lax.axis_index("c")
    sid = lax.axis_index("s")
    iota = lax.iota(jnp.int32, 16)

    @pl.when(cid == 0)
    def _():
        def initb(i, c):
            lastl[pl.ds(i * 16, 16)] = jnp.full((16,), -1, jnp.int32)
            return c
        lax.fori_loop(0, NPAD // 16, initb, 0)

        base = sid * EPT

        def chunk(ci, c0):
            pltpu.sync_copy(src_ref.at[pl.ds(base + ci * SCH, SCH)], sbuf)

            def scan(i, c):
                sv = sbuf[pl.ds(i * 16, 16)]
                key = sv * 16 + iota
                eidv = jnp.full((16,), base + ci * SCH, jnp.int32) + i * 16 + iota
                ks, vs = plsc.sort_key_val(key, eidv)
                srcs = lax.shift_right_logical(ks, 4)
                nxt = _shift_up1(srcs)
                msk = jnp.logical_or(srcs != nxt, iota == 15)
                plsc.store_scatter(lastl, [srcs], vs, mask=msk)
                return c
            lax.fori_loop(0, SCH // 16, scan, 0)
            return c0
        lax.fori_loop(0, EPT // SCH, chunk, 0)

        pltpu.sync_copy(lastl, shared.at[sid])
        plsc.subcore_barrier()
        for k in range(NCH):
            pltpu.sync_copy(shared.at[:, pl.ds(sid * NSL + k * 128, 128)], mrg)

            def red(q, c):
                acc = mrg[0, pl.ds(q * 16, 16)]
                for r in range(1, 16):
                    acc = jnp.maximum(acc, mrg[r, pl.ds(q * 16, 16)])
                leraw[pl.ds(k * 128 + q * 16, 16)] = acc
                lec[k, pl.ds(q * 16, 16)] = jnp.maximum(acc, 0)
                return c
            lax.fori_loop(0, 8, red, 0)

        pltpu.sync_copy(leraw, le_ref.at[pl.ds(sid * NSL, NSL)])
        cps = [pltpu.async_copy(dst_ref.at[lec.at[k]], d2b.at[k], sem)
               for k in range(NCH)]
        for cp in cps:
            cp.wait()
        cps = []
        for k in range(NCH):
            cps.append(pltpu.async_copy(et_ref.at[lec.at[k]], ttb.at[k], sem))
            cps.append(pltpu.async_copy(
                ea_ref.at[lec.at[k]], eab.at[pl.ds(k * 128, 128)], sem))
            cps.append(pltpu.async_copy(
                nf_ref.at[d2b.at[k]], nfb.at[pl.ds(k * 128, 128)], sem))
        for cp in cps:
            cp.wait()
        for k in range(NCH):
            pltpu.sync_copy(ttb.at[k],
                            tt_ref.at[pl.ds(sid * NSL + k * 128, 128)])
        pltpu.sync_copy(eab, eag_ref.at[pl.ds(sid * NSL, NSL)])
        pltpu.sync_copy(nfb, nfd2_ref.at[pl.ds(sid * NSL, NSL)])


def _last_edge(src, dst, et, ea, nf):
    mesh = plsc.VectorSubcoreMesh(core_axis_name="c", subcore_axis_name="s")
    f = pl.kernel(
        _last_sc_body,
        mesh=mesh,
        compiler_params=pltpu.CompilerParams(use_tc_tiling_on_sc=False,
                                             needs_layout_passes=False),
        out_type=[
            jax.ShapeDtypeStruct((NPAD,), jnp.int32),
            jax.ShapeDtypeStruct((NPAD,), jnp.float32),
            jax.ShapeDtypeStruct((NPAD, EDGE), jnp.float32),
            jax.ShapeDtypeStruct((NPAD, NODE), jnp.float32),
        ],
        scratch_types=[
            pltpu.VMEM((NPAD,), jnp.int32),
            pltpu.VMEM((SCH,), jnp.int32),
            pltpu.VMEM((16, 128), jnp.int32),
            pltpu.VMEM((NSL,), jnp.int32),
            pltpu.VMEM((NCH, 128), jnp.int32),
            pltpu.VMEM((NCH, 128), jnp.int32),
            pltpu.VMEM((NCH, 128), jnp.float32),
            pltpu.VMEM((NSL, EDGE), jnp.float32),
            pltpu.VMEM((NSL, NODE), jnp.float32),
            pltpu.VMEM_SHARED((16, NPAD), jnp.int32),
            pltpu.SemaphoreType.DMA,
        ],
    )
    return f(src, dst, et, ea, nf)


def kernel(node_features, edge_index, edge_attr, edge_times, time_w, time_b,
           msg_W1, msg_b1, msg_W2, msg_b2, gru_Wih, gru_Whh, gru_bih, gru_bhh,
           gat0_W, gat0_asrc, gat0_adst, gat0_b,
           gat1_W, gat1_asrc, gat1_adst, gat1_b,
           cls_W1, cls_b1, cls_W2, cls_b2):
    src = edge_index[0]
    dst = edge_index[1]
    lep, ttp, eap, nfd2p = _last_edge(src, dst, edge_times, edge_attr,
                                      node_features)
    has_f = (lep[:N] >= 0).astype(jnp.float32)[:, None]

    w1a = msg_W1[:NODE]
    w1c = msg_W1[NODE + MEM:NODE + MEM + NODE]
    w1d = msg_W1[NODE + MEM + NODE:NODE + MEM + NODE + EDGE]
    w1e = msg_W1[NODE + MEM + NODE + EDGE:]
    wiht = gru_Wih.T
    w0cat = jnp.transpose(gat0_W, (1, 0, 2)).reshape(L0_IN, HEADS * HPD)
    w1cat = jnp.transpose(gat1_W, (1, 0, 2)).reshape(NODE, HEADS * HPD)

    def amat(asrc, adst):
        a = jnp.zeros((HEADS * HPD, 2 * HEADS), jnp.float32)
        for h in range(HEADS):
            a = a.at[h * HPD:(h + 1) * HPD, h].set(asrc[h])
            a = a.at[h * HPD:(h + 1) * HPD, HEADS + h].set(adst[h])
        return a

    a0 = amat(gat0_asrc, gat0_adst)
    a1 = amat(gat1_asrc, gat1_adst)
    b0flat = gat0_b.reshape(1, -1)
    b1flat = gat1_b.reshape(1, -1)
    z = jnp.zeros((NPAD, WX), jnp.float32)

    grid = (N // BLK,)
    hhx0, sd0 = pl.pallas_call(
        _dense1_body,
        grid=grid,
        in_specs=[
            _rows(NODE), _rows(NODE), _rows(EDGE), _rows(1), _rows(1),
            _full((1, TIME)), _full((1, TIME)),
            _full((NODE, MSG)), _full((NODE, MSG)), _full((EDGE, MSG)),
            _full((TIME, MSG)), _full((1, MSG)), _full((MSG, MSG)),
            _full((1, MSG)), _full((MSG, 3 * MEM)), _full((1, 3 * MEM)),
            _full((1, 3 * MEM)), _full((L0_IN, HEADS * HPD)),
            _full((HEADS * HPD, 2 * HEADS)),
        ],
        out_specs=[_rows(WX), _rows(16)],
        out_shape=[
            jax.ShapeDtypeStruct((N, WX), jnp.float32),
            jax.ShapeDtypeStruct((N, 16), jnp.float32),
        ],
    )(node_features, nfd2p, eap, ttp[:, None], has_f,
      time_w[None, :], time_b[None, :], w1a, w1c, w1d, w1e,
      msg_b1[None, :], msg_W2, msg_b2[None, :],
      wiht, gru_bih[None, :], gru_bhh[None, :], w0cat, a0)

    acc0 = _gat_aggregate(src, dst, hhx0, sd0, z)

    acc_a = pl.BlockSpec((1, BLK, WX), lambda i: (0, i, 0))
    acc_b = pl.BlockSpec((1, BLK, WX), lambda i: (1, i, 0))

    hhx1, sd1 = pl.pallas_call(
        _finish_proj_body,
        grid=grid,
        in_specs=[acc_a, acc_b, _full((1, HEADS * HPD)),
                  _full((NODE, HEADS * HPD)), _full((HEADS * HPD, 2 * HEADS))],
        out_specs=[_rows(WX), _rows(16)],
        out_shape=[
            jax.ShapeDtypeStruct((N, WX), jnp.float32),
            jax.ShapeDtypeStruct((N, 16), jnp.float32),
        ],
    )(acc0, acc0, b0flat, w1cat, a1)

    acc1 = _gat_aggregate(src, dst, hhx1, sd1, z)

    logits = pl.pallas_call(
        _finish_cls_body,
        grid=grid,
        in_specs=[acc_a, acc_b, _full((1, HEADS * HPD)),
                  _full((NODE, NODE // 2)), _full((1, NODE // 2)),
                  _full((NODE // 2, 1)), _full((1, 1))],
        out_specs=_rows(1),
        out_shape=jax.ShapeDtypeStruct((N, 1), jnp.float32),
    )(acc1, acc1, b1flat,
      cls_W1, cls_b1[None, :], cls_W2, cls_b2[None, :])

    return logits


# speedup vs baseline: 1.9940x; 1.0453x over previous
import functools

import jax
import jax.numpy as jnp
from jax import lax
from jax.experimental import pallas as pl
from jax.experimental.pallas import tpu as pltpu
from jax.experimental.pallas import tpu_sc as plsc

N = 10000
E = 320000
NODE = 128
EDGE = 16
TIME = 32
MEM = 128
HEADS = 4
HPD = 32
MSG = NODE + EDGE + TIME
L0_IN = NODE + MEM + TIME

BLK = 1000
NW = 32
EPW = E // NW
KW = 80
NWIN = EPW // KW
NPAD = 10240
NPS = NPAD // 16
WX = 144
SS0 = NODE
DN0 = NODE + HEADS


def _pack_ext(hh, s):
    b = hh.shape[0]
    return jnp.concatenate(
        [hh, s[:, :HEADS], jnp.ones((b, HEADS), jnp.float32),
         jnp.zeros((b, WX - DN0 - HEADS), jnp.float32)], axis=1)


def _pack_sd(s):
    b = s.shape[0]
    return jnp.concatenate(
        [s[:, HEADS:], jnp.zeros((b, 12), jnp.float32)], axis=1)


def _dense1_body(nf_ref, nfd2_ref, ea_ref, tt_ref, has_ref, tw_ref, tb_ref,
                 w1a_ref, w1c_ref, w1d_ref, w1e_ref, b1_ref, w2_ref, b2_ref,
                 wiht_ref, bih_ref, bhh_ref, w0_ref, a0_ref,
                 hhx_ref, sd_ref):
    nf = nf_ref[...]
    t = tt_ref[...] * has_ref[...]
    te = jnp.sin(tw_ref[...] * t + tb_ref[...])
    u = (nf @ w1a_ref[...] + nfd2_ref[...] @ w1c_ref[...]
         + ea_ref[...] @ w1d_ref[...] + te @ w1e_ref[...] + b1_ref[...])
    u = jnp.maximum(u, 0.0)
    msgs = u @ w2_ref[...] + b2_ref[...]
    gi = msgs @ wiht_ref[...] + bih_ref[...]
    gh = bhh_ref[...]
    ir, iz, inn = gi[:, :MEM], gi[:, MEM:2 * MEM], gi[:, 2 * MEM:]
    hr, hz, hn = gh[:, :MEM], gh[:, MEM:2 * MEM], gh[:, 2 * MEM:]
    r = jax.nn.sigmoid(ir + hr)
    z = jax.nn.sigmoid(iz + hz)
    nn_ = jnp.tanh(inn + r * hn)
    newh = (1.0 - z) * nn_
    memv = newh * has_ref[...]
    x = jnp.concatenate([nf, memv, te], axis=1)
    hh0 = x @ w0_ref[...]
    s0 = hh0 @ a0_ref[...]
    hhx_ref[...] = _pack_ext(hh0, s0)
    sd_ref[...] = _pack_sd(s0)


def _unpack_finish(a0_ref, a1_ref, b_ref):
    acc = a0_ref[0] + a1_ref[0]
    parts = []
    for h in range(HEADS):
        parts.append(acc[:, h * HPD:(h + 1) * HPD]
                     / (acc[:, DN0 + h:DN0 + h + 1] + 1e-16))
    return jnp.concatenate(parts, axis=1) + b_ref[...]


def _finish_proj_body(a0_ref, a1_ref, b_ref, w_ref, am_ref, hhx_ref, sd_ref):
    x = _unpack_finish(a0_ref, a1_ref, b_ref)
    hh = x @ w_ref[...]
    s = hh @ am_ref[...]
    hhx_ref[...] = _pack_ext(hh, s)
    sd_ref[...] = _pack_sd(s)


def _finish_cls_body(a0_ref, a1_ref, b_ref, w1_ref, b1_ref, w2_ref, b2_ref,
                     out_ref):
    x = _unpack_finish(a0_ref, a1_ref, b_ref)
    u = jnp.maximum(x @ w1_ref[...] + b1_ref[...], 0.0)
    out_ref[...] = u @ w2_ref[...] + b2_ref[...]


def _full(shape):
    return pl.BlockSpec(shape, lambda i: (0,) * len(shape))


def _rows(width):
    return pl.BlockSpec((BLK, width), lambda i: (i, 0))


def _bcast_lane(vec, j):
    return lax.gather(
        vec, jnp.full((16, 1), j, jnp.int32),
        lax.GatherDimensionNumbers(offset_dims=(), collapsed_slice_dims=(0,),
                                   start_index_map=(0,)),
        (1,), mode=lax.GatherScatterMode.PROMISE_IN_BOUNDS)


def _shift4(vec):
    idx = jnp.maximum(lax.iota(jnp.int32, 16) - 4, 0)
    return lax.gather(
        vec, idx[:, None],
        lax.GatherDimensionNumbers(offset_dims=(), collapsed_slice_dims=(0,),
                                   start_index_map=(0,)),
        (1,), mode=lax.GatherScatterMode.PROMISE_IN_BOUNDS)


CHW = 64
CH_N = (NWIN + CHW - 1) // CHW
CHT = (NWIN - (CH_N - 1) * CHW)


def _gat_sc_body(src2_ref, dst2_ref, hhx_ref, sd_ref, z_ref,
                 acc_ref,
                 sbk, dbk, rows, sdb, out_sp,
                 sg0, sg1, ss0, ss1):
    cid = lax.axis_index("c")
    sid = lax.axis_index("s")
    wid = sid * 2 + cid
    sg = (sg0, sg1)
    ss = (ss0, ss1)

    pltpu.sync_copy(z_ref.at[pl.ds(sid * NPS, NPS)],
                    out_sp.at[pl.ds(sid * NPS, NPS)])
    plsc.subcore_barrier()

    base_row = wid * NWIN

    def src_at(wl):
        return sbk.at[wl]

    def start_gather(wl, b):
        pltpu.async_copy(hhx_ref.at[src_at(wl)], rows.at[b], sg[b])
        pltpu.async_copy(sd_ref.at[dbk.at[wl]], sdb.at[b], sg[b])

    def wait_gather(wl, b):
        pltpu.make_async_copy(hhx_ref.at[src_at(wl)], rows.at[b], sg[b]).wait()
        pltpu.make_async_copy(sd_ref.at[dbk.at[wl]], sdb.at[b], sg[b]).wait()

    def start_scatter(wl, b):
        pltpu.async_copy(rows.at[b], out_sp.at[dbk.at[wl]], ss[b], add=True)

    def wait_scatter(wl, b):
        pltpu.make_async_copy(rows.at[b], out_sp.at[dbk.at[wl]], ss[b]).wait()

    def compute(b):
        for j in range(KW):
            sseg = rows[b, j, pl.ds(SS0, 16)]
            v = sseg + sdb[b, j, :]
            v = jnp.where(v >= 0.0, v, 0.2 * v)
            w16 = jnp.exp(v)
            rows[b, j, pl.ds(SS0, 16)] = sseg * _shift4(w16)
            for h in range(HEADS):
                wh = _bcast_lane(w16, h)
                c0 = h * HPD
                rows[b, j, pl.ds(c0, 16)] = rows[b, j, pl.ds(c0, 16)] * wh
                rows[b, j, pl.ds(c0 + 16, 16)] = (
                    rows[b, j, pl.ds(c0 + 16, 16)] * wh)

    for ch in range(CH_N):
        nwc = CHW if ch < CH_N - 1 else CHT
        rb = base_row + ch * CHW
        ca = pltpu.async_copy(src2_ref.at[pl.ds(rb, nwc)],
                              sbk.at[pl.ds(0, nwc)], sg[0])
        cb = pltpu.async_copy(dst2_ref.at[pl.ds(rb, nwc)],
                              dbk.at[pl.ds(0, nwc)], sg[0])
        ca.wait()
        cb.wait()

        start_gather(0, 0)

        def step(i, carry):
            w0 = 2 * i
            @pl.when(i > 0)
            def _():
                wait_scatter(w0 - 1, 1)
            start_gather(w0 + 1, 1)
            wait_gather(w0, 0)
            compute(0)
            start_scatter(w0, 0)
            wait_scatter(w0, 0)
            start_gather(w0 + 2, 0)
            wait_gather(w0 + 1, 1)
            compute(1)
            start_scatter(w0 + 1, 1)
            return carry

        lax.fori_loop(0, (nwc - 1) // 2, step, 0)
        if nwc % 2:
            last = nwc - 1
            wait_scatter(last - 1, 1)
            wait_gather(last, 0)
            compute(0)
            start_scatter(last, 0)
            wait_scatter(last, 0)
        else:
            wait_scatter(nwc - 3, 1)
            wait_gather(nwc - 2, 0)
            compute(0)
            start_scatter(nwc - 2, 0)
            start_gather(nwc - 1, 1)
            wait_gather(nwc - 1, 1)
            compute(1)
            start_scatter(nwc - 1, 1)
            wait_scatter(nwc - 2, 0)
            wait_scatter(nwc - 1, 1)

    plsc.subcore_barrier()
    pltpu.sync_copy(out_sp.at[pl.ds(sid * NPS, NPS)],
                    acc_ref.at[cid, pl.ds(sid * NPS, NPS)])


def _gat_aggregate(src2, dst2, hhx, sd, z):
    mesh = plsc.VectorSubcoreMesh(core_axis_name="c", subcore_axis_name="s")
    f = pl.kernel(
        _gat_sc_body,
        mesh=mesh,
        compiler_params=pltpu.CompilerParams(use_tc_tiling_on_sc=False,
                                             needs_layout_passes=False),
        out_type=jax.ShapeDtypeStruct((2, NPAD, WX), jnp.float32),
        scratch_types=[
            pltpu.VMEM((CHW, KW), jnp.int32),
            pltpu.VMEM((CHW, KW), jnp.int32),
            pltpu.VMEM((2, KW, WX), jnp.float32),
            pltpu.VMEM((2, KW, 16), jnp.float32),
            pltpu.VMEM_SHARED((NPAD, WX), jnp.float32),
            pltpu.SemaphoreType.DMA,
            pltpu.SemaphoreType.DMA,
            pltpu.SemaphoreType.DMA,
            pltpu.SemaphoreType.DMA,
        ],
    )
    return f(src2, dst2, hhx, sd, z)


def _shift_up1(vec):
    idx = jnp.minimum(lax.iota(jnp.int32, 16) + 1, 15)
    return lax.gather(
        vec, idx[:, None],
        lax.GatherDimensionNumbers(offset_dims=(), collapsed_slice_dims=(0,),
                                   start_index_map=(0,)),
        (1,), mode=lax.GatherScatterMode.PROMISE_IN_BOUNDS)


EPT = E // 16
SCH = 4000
NSL = NPAD // 16
NCH = NSL // 128


def _last_sc_body(src_ref, dst_ref, et_ref, ea_ref, nf_ref,
                  le_ref, tt_ref, eag_ref, nfd2_ref,
                  lastl, sbuf, mrg, leraw, lec, d2b, ttb, eab, nfb,
                  shared, sem):
    cid = lax.axis_index("c")
    sid = lax.axis_index("s")
    iota = lax.iota(jnp.int32, 16)

    @pl.when(cid == 0)
    def _():
        def initb(i, c):
            lastl[pl.ds(i * 16, 16)] = jnp.full((16,), -1, jnp.int32)
            return c
        lax.fori_loop(0, NPAD // 16, initb, 0)

        base = sid * EPT

        def chunk(ci, c0):
            pltpu.sync_copy(src_ref.at[pl.ds(base + ci * SCH, SCH)], sbuf)

            def scan(i, c):
                sv = sbuf[pl.ds(i * 16, 16)]
                key = sv * 16 + iota
                eidv = jnp.full((16,), base + ci * SCH, jnp.int32) + i * 16 + iota
                ks, vs = plsc.sort_key_val(key, eidv)
                srcs = lax.shift_right_logical(ks, 4)
                nxt = _shift_up1(srcs)
                msk = jnp.logical_or(srcs != nxt, iota == 15)
                plsc.store_scatter(lastl, [srcs], vs, mask=msk)
                return c
            lax.fori_loop(0, SCH // 16, scan, 0)
            return c0
        lax.fori_loop(0, EPT // SCH, chunk, 0)

        pltpu.sync_copy(lastl, shared.at[sid])
        plsc.subcore_barrier()
        for k in range(NCH):
            pltpu.sync_copy(shared.at[:, pl.ds(sid * NSL + k * 128, 128)], mrg)

            def red(q, c):
                acc = mrg[0, pl.ds(q * 16, 16)]
                for r in range(1, 16):
                    acc = jnp.maximum(acc, mrg[r, pl.ds(q * 16, 16)])
                leraw[pl.ds(k * 128 + q * 16, 16)] = acc
                lec[k, pl.ds(q * 16, 16)] = jnp.maximum(acc, 0)
                return c
            lax.fori_loop(0, 8, red, 0)

        pltpu.sync_copy(leraw, le_ref.at[pl.ds(sid * NSL, NSL)])
        cps = [pltpu.async_copy(dst_ref.at[lec.at[k]], d2b.at[k], sem)
               for k in range(NCH)]
        for cp in cps:
            cp.wait()
        cps = []
        for k in range(NCH):
            cps.append(pltpu.async_copy(et_ref.at[lec.at[k]], ttb.at[k], sem))
            cps.append(pltpu.async_copy(
                ea_ref.at[lec.at[k]], eab.at[pl.ds(k * 128, 128)], sem))
            cps.append(pltpu.async_copy(
                nf_ref.at[d2b.at[k]], nfb.at[pl.ds(k * 128, 128)], sem))
        for cp in cps:
            cp.wait()
        for k in range(NCH):
            pltpu.sync_copy(ttb.at[k],
                            tt_ref.at[pl.ds(sid * NSL + k * 128, 128)])
        pltpu.sync_copy(eab, eag_ref.at[pl.ds(sid * NSL, NSL)])
        pltpu.sync_copy(nfb, nfd2_ref.at[pl.ds(sid * NSL, NSL)])


def _last_edge(src, dst, et, ea, nf):
    mesh = plsc.VectorSubcoreMesh(core_axis_name="c", subcore_axis_name="s")
    f = pl.kernel(
        _last_sc_body,
        mesh=mesh,
        compiler_params=pltpu.CompilerParams(use_tc_tiling_on_sc=False,
                                             needs_layout_passes=False),
        out_type=[
            jax.ShapeDtypeStruct((NPAD,), jnp.int32),
            jax.ShapeDtypeStruct((NPAD,), jnp.float32),
            jax.ShapeDtypeStruct((NPAD, EDGE), jnp.float32),
            jax.ShapeDtypeStruct((NPAD, NODE), jnp.float32),
        ],
        scratch_types=[
            pltpu.VMEM((NPAD,), jnp.int32),
            pltpu.VMEM((SCH,), jnp.int32),
            pltpu.VMEM((16, 128), jnp.int32),
            pltpu.VMEM((NSL,), jnp.int32),
            pltpu.VMEM((NCH, 128), jnp.int32),
            pltpu.VMEM((NCH, 128), jnp.int32),
            pltpu.VMEM((NCH, 128), jnp.float32),
            pltpu.VMEM((NSL, EDGE), jnp.float32),
            pltpu.VMEM((NSL, NODE), jnp.float32),
            pltpu.VMEM_SHARED((16, NPAD), jnp.int32),
            pltpu.SemaphoreType.DMA,
        ],
    )
    return f(src, dst, et, ea, nf)


def kernel(node_features, edge_index, edge_attr, edge_times, time_w, time_b,
           msg_W1, msg_b1, msg_W2, msg_b2, gru_Wih, gru_Whh, gru_bih, gru_bhh,
           gat0_W, gat0_asrc, gat0_adst, gat0_b,
           gat1_W, gat1_asrc, gat1_adst, gat1_b,
           cls_W1, cls_b1, cls_W2, cls_b2):
    src = edge_index[0]
    dst = edge_index[1]
    lep, ttp, eap, nfd2p = _last_edge(src, dst, edge_times, edge_attr,
                                      node_features)
    has_f = (lep[:N] >= 0).astype(jnp.float32)[:, None]

    w1a = msg_W1[:NODE]
    w1c = msg_W1[NODE + MEM:NODE + MEM + NODE]
    w1d = msg_W1[NODE + MEM + NODE:NODE + MEM + NODE + EDGE]
    w1e = msg_W1[NODE + MEM + NODE + EDGE:]
    wiht = gru_Wih.T
    w0cat = jnp.transpose(gat0_W, (1, 0, 2)).reshape(L0_IN, HEADS * HPD)
    w1cat = jnp.transpose(gat1_W, (1, 0, 2)).reshape(NODE, HEADS * HPD)

    def amat(asrc, adst):
        a = jnp.zeros((HEADS * HPD, 2 * HEADS), jnp.float32)
        for h in range(HEADS):
            a = a.at[h * HPD:(h + 1) * HPD, h].set(asrc[h])
            a = a.at[h * HPD:(h + 1) * HPD, HEADS + h].set(adst[h])
        return a

    a0 = amat(gat0_asrc, gat0_adst)
    a1 = amat(gat1_asrc, gat1_adst)
    b0flat = gat0_b.reshape(1, -1)
    b1flat = gat1_b.reshape(1, -1)
    z = jnp.zeros((NPAD, WX), jnp.float32)

    grid = (N // BLK,)
    hhx0, sd0 = pl.pallas_call(
        _dense1_body,
        grid=grid,
        in_specs=[
            _rows(NODE), _rows(NODE), _rows(EDGE), _rows(1), _rows(1),
            _full((1, TIME)), _full((1, TIME)),
            _full((NODE, MSG)), _full((NODE, MSG)), _full((EDGE, MSG)),
            _full((TIME, MSG)), _full((1, MSG)), _full((MSG, MSG)),
            _full((1, MSG)), _full((MSG, 3 * MEM)), _full((1, 3 * MEM)),
            _full((1, 3 * MEM)), _full((L0_IN, HEADS * HPD)),
            _full((HEADS * HPD, 2 * HEADS)),
        ],
        out_specs=[_rows(WX), _rows(16)],
        out_shape=[
            jax.ShapeDtypeStruct((N, WX), jnp.float32),
            jax.ShapeDtypeStruct((N, 16), jnp.float32),
        ],
    )(node_features, nfd2p, eap, ttp[:, None], has_f,
      time_w[None, :], time_b[None, :], w1a, w1c, w1d, w1e,
      msg_b1[None, :], msg_W2, msg_b2[None, :],
      wiht, gru_bih[None, :], gru_bhh[None, :], w0cat, a0)

    src2 = src.reshape(E // KW, KW)
    dst2 = dst.reshape(E // KW, KW)
    acc0 = _gat_aggregate(src2, dst2, hhx0, sd0, z)

    acc_a = pl.BlockSpec((1, BLK, WX), lambda i: (0, i, 0))
    acc_b = pl.BlockSpec((1, BLK, WX), lambda i: (1, i, 0))

    hhx1, sd1 = pl.pallas_call(
        _finish_proj_body,
        grid=grid,
        in_specs=[acc_a, acc_b, _full((1, HEADS * HPD)),
                  _full((NODE, HEADS * HPD)), _full((HEADS * HPD, 2 * HEADS))],
        out_specs=[_rows(WX), _rows(16)],
        out_shape=[
            jax.ShapeDtypeStruct((N, WX), jnp.float32),
            jax.ShapeDtypeStruct((N, 16), jnp.float32),
        ],
    )(acc0, acc0, b0flat, w1cat, a1)

    acc1 = _gat_aggregate(src2, dst2, hhx1, sd1, z)

    logits = pl.pallas_call(
        _finish_cls_body,
        grid=grid,
        in_specs=[acc_a, acc_b, _full((1, HEADS * HPD)),
                  _full((NODE, NODE // 2)), _full((1, NODE // 2)),
                  _full((NODE // 2, 1)), _full((1, 1))],
        out_specs=_rows(1),
        out_shape=jax.ShapeDtypeStruct((N, 1), jnp.float32),
    )(acc1, acc1, b1flat,
      cls_W1, cls_b1[None, :], cls_W2, cls_b2[None, :])

    return logits
